# baseline (speedup 1.0000x reference)
import functools

import jax
import jax.numpy as jnp
from jax import lax
from jax.experimental import pallas as pl
from jax.experimental.pallas import tpu as pltpu
from jax.experimental.pallas import tpu_sc as plsc

_N = 10000
_E = 320000
_D = 128
_G = 16

_NC = 2
_NS = 16
_NW = _NC * _NS
_CH = 64
_CPT = 160
_HC = 40
_NBUF = 4
_LG = 2
_EPT = _CPT * _CH
_E_PAD = _NW * _EPT
_N_ACC = 10112
_RPT = _N_ACC // _NS
_DUMP = _N

_R = 1000
_NB = _N // _R


def _sc_scatter_sum(h, src2d, dst2d, zrows):
    mesh = plsc.VectorSubcoreMesh(core_axis_name="c", subcore_axis_name="s")

    @functools.partial(
        pl.kernel,
        out_type=jax.ShapeDtypeStruct((_NC * _N_ACC, _D), jnp.float32),
        mesh=mesh,
        scratch_types=[
            pltpu.VMEM((_HC, _CH), jnp.int32),
            pltpu.VMEM((_HC, _CH), jnp.int32),
            pltpu.VMEM((_NBUF, _CH, _D), jnp.float32),
            pltpu.VMEM_SHARED((_N_ACC, _D), jnp.float32),
            pltpu.SemaphoreType.DMA,
            pltpu.SemaphoreType.DMA,
            pltpu.SemaphoreType.DMA,
            pltpu.SemaphoreType.DMA,
            pltpu.SemaphoreType.DMA,
            pltpu.SemaphoreType.DMA,
            pltpu.SemaphoreType.DMA,
            pltpu.SemaphoreType.DMA,
        ],
    )
    def k(h_hbm, src_hbm, dst_hbm, z_hbm, out_hbm,
          src_v, dst_v, rows_v, acc_sh,
          sem0, sem1, sem2, sem3, sem4, sem5, sem6, sem7):
        c = lax.axis_index("c")
        s = lax.axis_index("s")
        wid = c * _NS + s
        pltpu.sync_copy(z_hbm, acc_sh.at[pl.ds(s * _RPT, _RPT)])
        plsc.subcore_barrier()

        gsems = (sem0, sem1, sem2, sem3)
        ssems = (sem4, sem5, sem6, sem7)

        def gather(j, b):
            pltpu.async_copy(h_hbm.at[src_v.at[j]], rows_v.at[b], gsems[b])

        def gwait(j, b):
            pltpu.make_async_copy(
                h_hbm.at[src_v.at[j]], rows_v.at[b], gsems[b]).wait()

        def scat(j, b):
            pltpu.async_copy(rows_v.at[b], acc_sh.at[dst_v.at[j]],
                             ssems[b], add=True)

        def swait(j, b):
            pltpu.make_async_copy(
                rows_v.at[b], acc_sh.at[dst_v.at[j]], ssems[b]).wait()

        for half in range(_CPT // _HC):
            base = wid * _CPT + half * _HC
            pltpu.sync_copy(src_hbm.at[pl.ds(base, _HC)], src_v)
            pltpu.sync_copy(dst_hbm.at[pl.ds(base, _HC)], dst_v)

            def step(j, b):
                gwait(j, b)
                scat(j, b)
                nb = (b + _LG) % _NBUF
                swait(j + _LG - _NBUF, nb)
                gather(j + _LG, nb)

            for j in range(_LG):
                gather(j, j)
            for j in range(_NBUF):
                gwait(j, j)
                scat(j, j)
                if j + _LG < _NBUF:
                    gather(j + _LG, j + _LG)
                else:
                    swait(j + _LG - _NBUF, (j + _LG) % _NBUF)
                    gather(j + _LG, (j + _LG) % _NBUF)

            def body(g, carry):
                for b in range(_NBUF):
                    step(_NBUF * g + b, b)
                return carry

            lax.fori_loop(1, _HC // _NBUF - 1, body, 0)
            for j in range(_HC - _NBUF, _HC):
                gwait(j, j % _NBUF)
                scat(j, j % _NBUF)
                if j + _LG < _HC:
                    swait(j + _LG - _NBUF, (j + _LG) % _NBUF)
                    gather(j + _LG, (j + _LG) % _NBUF)
            for j in range(_HC - _NBUF, _HC):
                swait(j, j % _NBUF)

        plsc.subcore_barrier()
        row0 = c * _N_ACC + s * _RPT
        pltpu.sync_copy(acc_sh.at[pl.ds(s * _RPT, _RPT)],
                        out_hbm.at[pl.ds(row0, _RPT)])

    return k(h, src2d, dst2d, zrows)


def _dense_body(a0_ref, a1_ref, w_ref, b_ref, z_ref, st_ref, acc_ref):
    i = pl.program_id(0)
    a = a0_ref[0] + a1_ref[0]
    z = lax.dot_general(a, w_ref[...], (((1,), (1,)), ((), ())),
                        preferred_element_type=jnp.float32) + b_ref[...]
    z_ref[...] = z

    @pl.when(i == 0)
    def _init():
        acc_ref[...] = jnp.zeros_like(acc_ref)

    acc_ref[0:1, :] += jnp.sum(z, axis=0, keepdims=True)
    acc_ref[1:2, :] += jnp.sum(z * z, axis=0, keepdims=True)

    @pl.when(i == _NB - 1)
    def _fin():
        st_ref[...] = acc_ref[...]


def _tc_linear_stats(a2, W, b):
    return pl.pallas_call(
        _dense_body,
        grid=(_NB,),
        in_specs=[
            pl.BlockSpec((1, _R, _D), lambda i: (0, i, 0)),
            pl.BlockSpec((1, _R, _D), lambda i: (1, i, 0)),
            pl.BlockSpec((_D, _D), lambda i: (0, 0)),
            pl.BlockSpec((1, _D), lambda i: (0, 0)),
        ],
        out_specs=[
            pl.BlockSpec((_R, _D), lambda i: (i, 0)),
            pl.BlockSpec((2, _D), lambda i: (0, 0)),
        ],
        out_shape=[
            jax.ShapeDtypeStruct((_N, _D), jnp.float32),
            jax.ShapeDtypeStruct((2, _D), jnp.float32),
        ],
        scratch_shapes=[pltpu.VMEM((2, _D), jnp.float32)],
    )(a2, a2, W, b)


def _bn_elu(z, st_ref, g_ref, be_ref):
    mu = st_ref[0:1, :] * (1.0 / _N)
    var = st_ref[1:2, :] * (1.0 / _N) - mu * mu
    inv = lax.rsqrt(var + 1e-5)
    y = (z - mu) * (inv * g_ref[...]) + be_ref[...]
    return jnp.where(y > 0, y, jnp.exp(y) - 1.0)


def _norm_body(z_ref, st_ref, g_ref, be_ref, h_ref):
    h_ref[...] = _bn_elu(z_ref[...], st_ref, g_ref, be_ref)


def _tc_norm(z, st, g, be):
    return pl.pallas_call(
        _norm_body,
        grid=(_NB,),
        in_specs=[
            pl.BlockSpec((_R, _D), lambda i: (i, 0)),
            pl.BlockSpec((2, _D), lambda i: (0, 0)),
            pl.BlockSpec((1, _D), lambda i: (0, 0)),
            pl.BlockSpec((1, _D), lambda i: (0, 0)),
        ],
        out_specs=pl.BlockSpec((_R, _D), lambda i: (i, 0)),
        out_shape=jax.ShapeDtypeStruct((_N, _D), jnp.float32),
    )(z, st, g, be)


def _entropy(h, wc_ref, bc_ref):
    logits = lax.dot_general(h, wc_ref[...], (((1,), (1,)), ((), ())),
                             preferred_element_type=jnp.float32) + bc_ref[...]
    m = jnp.max(logits, axis=1, keepdims=True)
    lse = m + jnp.log(jnp.sum(jnp.exp(logits - m), axis=1, keepdims=True))
    logp = logits - lse
    return -jnp.sum(jnp.exp(logp) * logp, axis=1, keepdims=True)


def _norm3_body(z_ref, st_ref, g_ref, be_ref, wc_ref, bc_ref,
                h_ref, mx_ref, macc_ref):
    i = pl.program_id(0)
    h = _bn_elu(z_ref[...], st_ref, g_ref, be_ref)
    h_ref[...] = h
    hent = _entropy(h, wc_ref, bc_ref)
    bm = jnp.max(hent, axis=0, keepdims=True)

    @pl.when(i == 0)
    def _init():
        macc_ref[...] = jnp.full((1, 1), -jnp.inf, jnp.float32)

    macc_ref[...] = jnp.maximum(macc_ref[...], bm)

    @pl.when(i == _NB - 1)
    def _fin():
        mx_ref[...] = macc_ref[...]


def _tc_norm3(z, st, g, be, Wc, bc):
    return pl.pallas_call(
        _norm3_body,
        grid=(_NB,),
        in_specs=[
            pl.BlockSpec((_R, _D), lambda i: (i, 0)),
            pl.BlockSpec((2, _D), lambda i: (0, 0)),
            pl.BlockSpec((1, _D), lambda i: (0, 0)),
            pl.BlockSpec((1, _D), lambda i: (0, 0)),
            pl.BlockSpec((10, _D), lambda i: (0, 0)),
            pl.BlockSpec((1, 10), lambda i: (0, 0)),
        ],
        out_specs=[
            pl.BlockSpec((_R, _D), lambda i: (i, 0)),
            pl.BlockSpec((1, 1), lambda i: (0, 0)),
        ],
        out_shape=[
            jax.ShapeDtypeStruct((_N, _D), jnp.float32),
            jax.ShapeDtypeStruct((1, 1), jnp.float32),
        ],
        scratch_shapes=[pltpu.VMEM((1, 1), jnp.float32)],
    )(z, st, g, be, Wc, bc)


def _head_body(h_ref, mx_ref, gid_ref, wc_ref, bc_ref, t_ref, pool_ref):
    i = pl.program_id(0)
    h = h_ref[...]
    hent = _entropy(h, wc_ref, bc_ref)
    lam = 1.0 - hent / mx_ref[...]
    wgt = lam * h
    gid = gid_ref[0, 0, :]
    oh = (lax.broadcasted_iota(jnp.int32, (_G, _R), 0) == gid[None, :])
    part = lax.dot_general(oh.astype(jnp.float32), wgt,
                           (((1,), (0,)), ((), ())),
                           preferred_element_type=jnp.float32)

    @pl.when(i == 0)
    def _init():
        pool_ref[...] = jnp.zeros_like(pool_ref)

    pool_ref[...] += part

    @pl.when(i == _NB - 1)
    def _fin():
        t_ref[...] = lax.dot_general(
            pool_ref[...], wc_ref[...], (((1,), (1,)), ((), ())),
            preferred_element_type=jnp.float32) + bc_ref[...]


def _tc_head(h, hmax, gid3, Wc, bc):
    return pl.pallas_call(
        _head_body,
        grid=(_NB,),
        in_specs=[
            pl.BlockSpec((_R, _D), lambda i: (i, 0)),
            pl.BlockSpec((1, 1), lambda i: (0, 0)),
            pl.BlockSpec((1, 1, _R), lambda i: (i, 0, 0)),
            pl.BlockSpec((10, _D), lambda i: (0, 0)),
            pl.BlockSpec((1, 10), lambda i: (0, 0)),
        ],
        out_specs=pl.BlockSpec((_G, 10), lambda i: (0, 0)),
        out_shape=jax.ShapeDtypeStruct((_G, 10), jnp.float32),
        scratch_shapes=[pltpu.VMEM((_G, _D), jnp.float32)],
    )(h, hmax, gid3, Wc, bc)


def kernel(x, edge_index, graph_ids, W0, b0, g0, be0, W1, b1, g1, be1,
           W2, b2, g2, be2, Wc, bc):
    pad = _E_PAD - _E
    src2d = jnp.concatenate(
        [edge_index[0], jnp.zeros((pad,), jnp.int32)]).reshape(_NW * _CPT, _CH)
    pad_dst = _DUMP + jnp.arange(pad, dtype=jnp.int32) % (_N_ACC - _N)
    dst2d = jnp.concatenate(
        [edge_index[1], pad_dst]).reshape(_NW * _CPT, _CH)
    zrows = jnp.zeros((_RPT, _D), jnp.float32)
    gid3 = graph_ids.reshape(_NB, 1, _R)
    bc2 = bc.reshape(1, 10)

    h = x
    for (W, b, gm, be) in ((W0, b0, g0, be0), (W1, b1, g1, be1)):
        a2 = _sc_scatter_sum(h, src2d, dst2d, zrows).reshape(_NC, _N_ACC, _D)
        z, st = _tc_linear_stats(a2, W, b.reshape(1, _D))
        h = _tc_norm(z, st, gm.reshape(1, _D), be.reshape(1, _D))

    a2 = _sc_scatter_sum(h, src2d, dst2d, zrows).reshape(_NC, _N_ACC, _D)
    z, st = _tc_linear_stats(a2, W2, b2.reshape(1, _D))
    h, hmax = _tc_norm3(z, st, g2.reshape(1, _D), be2.reshape(1, _D), Wc, bc2)

    return _tc_head(h, hmax, gid3, Wc, bc2)


# speedup vs baseline: 2.8490x; 2.8490x over previous
import functools

import jax
import jax.numpy as jnp
from jax import lax
from jax.experimental import pallas as pl
from jax.experimental.pallas import tpu as pltpu
from jax.experimental.pallas import tpu_sc as plsc

_N = 10000
_E = 320000
_D = 128
_G = 16

_NC = 2
_NS = 16
_NW = _NC * _NS
_CH = 64
_CPT = 160
_HC = 40
_NBUF = 4
_LG = 2
_EPT = _CPT * _CH
_E_PAD = _NW * _EPT
_N_ACC = 10112
_RPT = _N_ACC // _NS
_DUMP = _N

_R = 1000
_NB = _N // _R


def _sc_scatter_sum(h, src2d, dst2d, zrows):
    mesh = plsc.VectorSubcoreMesh(core_axis_name="c", subcore_axis_name="s")

    @functools.partial(
        pl.kernel,
        out_type=jax.ShapeDtypeStruct((_NC * _N_ACC, _D), jnp.float32),
        mesh=mesh,
        scratch_types=[
            pltpu.VMEM((_HC, _CH), jnp.int32),
            pltpu.VMEM((_HC, _CH), jnp.int32),
            pltpu.VMEM((_NBUF, _CH, _D), jnp.float32),
            pltpu.VMEM_SHARED((_N_ACC, _D), jnp.float32),
            pltpu.SemaphoreType.DMA,
            pltpu.SemaphoreType.DMA,
            pltpu.SemaphoreType.DMA,
            pltpu.SemaphoreType.DMA,
            pltpu.SemaphoreType.DMA,
            pltpu.SemaphoreType.DMA,
            pltpu.SemaphoreType.DMA,
            pltpu.SemaphoreType.DMA,
        ],
    )
    def k(h_hbm, src_hbm, dst_hbm, z_hbm, out_hbm,
          src_v, dst_v, rows_v, acc_sh,
          sem0, sem1, sem2, sem3, sem4, sem5, sem6, sem7):
        c = lax.axis_index("c")
        s = lax.axis_index("s")
        wid = c * _NS + s
        pltpu.sync_copy(z_hbm.at[pl.ds(s * _RPT, _RPT)],
                        acc_sh.at[pl.ds(s * _RPT, _RPT)])
        plsc.subcore_barrier()

        gsems = (sem0, sem1, sem2, sem3)
        ssems = (sem4, sem5, sem6, sem7)

        def gather(j, b):
            pltpu.async_copy(h_hbm.at[src_v.at[j]], rows_v.at[b], gsems[b])

        def gwait(j, b):
            pltpu.make_async_copy(
                h_hbm.at[src_v.at[j]], rows_v.at[b], gsems[b]).wait()

        def scat(j, b):
            pltpu.async_copy(rows_v.at[b], acc_sh.at[dst_v.at[j]],
                             ssems[b], add=True)

        def swait(j, b):
            pltpu.make_async_copy(
                rows_v.at[b], acc_sh.at[dst_v.at[j]], ssems[b]).wait()

        for half in range(_CPT // _HC):
            base = wid * _CPT + half * _HC
            pltpu.sync_copy(src_hbm.at[pl.ds(base, _HC)], src_v)
            pltpu.sync_copy(dst_hbm.at[pl.ds(base, _HC)], dst_v)

            def step(j, b):
                gwait(j, b)
                scat(j, b)
                nb = (b + _LG) % _NBUF
                swait(j + _LG - _NBUF, nb)
                gather(j + _LG, nb)

            for j in range(_LG):
                gather(j, j)
            for j in range(_NBUF):
                gwait(j, j)
                scat(j, j)
                if j + _LG < _NBUF:
                    gather(j + _LG, j + _LG)
                else:
                    swait(j + _LG - _NBUF, (j + _LG) % _NBUF)
                    gather(j + _LG, (j + _LG) % _NBUF)

            def body(g, carry):
                for b in range(_NBUF):
                    step(_NBUF * g + b, b)
                return carry

            lax.fori_loop(1, _HC // _NBUF - 1, body, 0)
            for j in range(_HC - _NBUF, _HC):
                gwait(j, j % _NBUF)
                scat(j, j % _NBUF)
                if j + _LG < _HC:
                    swait(j + _LG - _NBUF, (j + _LG) % _NBUF)
                    gather(j + _LG, (j + _LG) % _NBUF)
            for j in range(_HC - _NBUF, _HC):
                swait(j, j % _NBUF)

        plsc.subcore_barrier()
        row0 = c * _N_ACC + s * _RPT
        pltpu.sync_copy(acc_sh.at[pl.ds(s * _RPT, _RPT)],
                        out_hbm.at[pl.ds(row0, _RPT)])

    return k(h, src2d, dst2d, zrows)


def _dense_body(a0_ref, a1_ref, w_ref, b_ref, z_ref, st_ref, acc_ref):
    i = pl.program_id(0)
    a = a0_ref[0] + a1_ref[0]
    z = lax.dot_general(a, w_ref[...], (((1,), (1,)), ((), ())),
                        preferred_element_type=jnp.float32) + b_ref[...]
    z_ref[...] = z

    @pl.when(i == 0)
    def _init():
        acc_ref[...] = jnp.zeros_like(acc_ref)

    acc_ref[0:1, :] += jnp.sum(z, axis=0, keepdims=True)
    acc_ref[1:2, :] += jnp.sum(z * z, axis=0, keepdims=True)

    @pl.when(i == _NB - 1)
    def _fin():
        st_ref[...] = acc_ref[...]


def _tc_linear_stats(a2, W, b):
    return pl.pallas_call(
        _dense_body,
        grid=(_NB,),
        in_specs=[
            pl.BlockSpec((1, _R, _D), lambda i: (0, i, 0)),
            pl.BlockSpec((1, _R, _D), lambda i: (1, i, 0)),
            pl.BlockSpec((_D, _D), lambda i: (0, 0)),
            pl.BlockSpec((1, _D), lambda i: (0, 0)),
        ],
        out_specs=[
            pl.BlockSpec((_R, _D), lambda i: (i, 0)),
            pl.BlockSpec((2, _D), lambda i: (0, 0)),
        ],
        out_shape=[
            jax.ShapeDtypeStruct((_N, _D), jnp.float32),
            jax.ShapeDtypeStruct((2, _D), jnp.float32),
        ],
        scratch_shapes=[pltpu.VMEM((2, _D), jnp.float32)],
    )(a2, a2, W, b)


def _bn_elu(z, st_ref, g_ref, be_ref):
    mu = st_ref[0:1, :] * (1.0 / _N)
    var = st_ref[1:2, :] * (1.0 / _N) - mu * mu
    inv = lax.rsqrt(var + 1e-5)
    y = (z - mu) * (inv * g_ref[...]) + be_ref[...]
    return jnp.where(y > 0, y, jnp.exp(y) - 1.0)


def _norm_body(z_ref, st_ref, g_ref, be_ref, h_ref):
    h_ref[...] = _bn_elu(z_ref[...], st_ref, g_ref, be_ref)


def _tc_norm(z, st, g, be):
    return pl.pallas_call(
        _norm_body,
        grid=(_NB,),
        in_specs=[
            pl.BlockSpec((_R, _D), lambda i: (i, 0)),
            pl.BlockSpec((2, _D), lambda i: (0, 0)),
            pl.BlockSpec((1, _D), lambda i: (0, 0)),
            pl.BlockSpec((1, _D), lambda i: (0, 0)),
        ],
        out_specs=pl.BlockSpec((_R, _D), lambda i: (i, 0)),
        out_shape=jax.ShapeDtypeStruct((_N, _D), jnp.float32),
    )(z, st, g, be)


def _entropy(h, wc_ref, bc_ref):
    logits = lax.dot_general(h, wc_ref[...], (((1,), (1,)), ((), ())),
                             preferred_element_type=jnp.float32) + bc_ref[...]
    m = jnp.max(logits, axis=1, keepdims=True)
    lse = m + jnp.log(jnp.sum(jnp.exp(logits - m), axis=1, keepdims=True))
    logp = logits - lse
    return -jnp.sum(jnp.exp(logp) * logp, axis=1, keepdims=True)


def _norm3_body(z_ref, st_ref, g_ref, be_ref, wc_ref, bc_ref,
                h_ref, mx_ref, macc_ref):
    i = pl.program_id(0)
    h = _bn_elu(z_ref[...], st_ref, g_ref, be_ref)
    h_ref[...] = h
    hent = _entropy(h, wc_ref, bc_ref)
    bm = jnp.max(hent, axis=0, keepdims=True)

    @pl.when(i == 0)
    def _init():
        macc_ref[...] = jnp.full((1, 1), -jnp.inf, jnp.float32)

    macc_ref[...] = jnp.maximum(macc_ref[...], bm)

    @pl.when(i == _NB - 1)
    def _fin():
        mx_ref[...] = macc_ref[...]


def _tc_norm3(z, st, g, be, Wc, bc):
    return pl.pallas_call(
        _norm3_body,
        grid=(_NB,),
        in_specs=[
            pl.BlockSpec((_R, _D), lambda i: (i, 0)),
            pl.BlockSpec((2, _D), lambda i: (0, 0)),
            pl.BlockSpec((1, _D), lambda i: (0, 0)),
            pl.BlockSpec((1, _D), lambda i: (0, 0)),
            pl.BlockSpec((10, _D), lambda i: (0, 0)),
            pl.BlockSpec((1, 10), lambda i: (0, 0)),
        ],
        out_specs=[
            pl.BlockSpec((_R, _D), lambda i: (i, 0)),
            pl.BlockSpec((1, 1), lambda i: (0, 0)),
        ],
        out_shape=[
            jax.ShapeDtypeStruct((_N, _D), jnp.float32),
            jax.ShapeDtypeStruct((1, 1), jnp.float32),
        ],
        scratch_shapes=[pltpu.VMEM((1, 1), jnp.float32)],
    )(z, st, g, be, Wc, bc)


def _head_body(h_ref, mx_ref, gid_ref, wc_ref, bc_ref, t_ref, pool_ref):
    i = pl.program_id(0)
    h = h_ref[...]
    hent = _entropy(h, wc_ref, bc_ref)
    lam = 1.0 - hent / mx_ref[...]
    wgt = lam * h
    gid = gid_ref[0, 0, :]
    oh = (lax.broadcasted_iota(jnp.int32, (_G, _R), 0) == gid[None, :])
    part = lax.dot_general(oh.astype(jnp.float32), wgt,
                           (((1,), (0,)), ((), ())),
                           preferred_element_type=jnp.float32)

    @pl.when(i == 0)
    def _init():
        pool_ref[...] = jnp.zeros_like(pool_ref)

    pool_ref[...] += part

    @pl.when(i == _NB - 1)
    def _fin():
        t_ref[...] = lax.dot_general(
            pool_ref[...], wc_ref[...], (((1,), (1,)), ((), ())),
            preferred_element_type=jnp.float32) + bc_ref[...]


def _tc_head(h, hmax, gid3, Wc, bc):
    return pl.pallas_call(
        _head_body,
        grid=(_NB,),
        in_specs=[
            pl.BlockSpec((_R, _D), lambda i: (i, 0)),
            pl.BlockSpec((1, 1), lambda i: (0, 0)),
            pl.BlockSpec((1, 1, _R), lambda i: (i, 0, 0)),
            pl.BlockSpec((10, _D), lambda i: (0, 0)),
            pl.BlockSpec((1, 10), lambda i: (0, 0)),
        ],
        out_specs=pl.BlockSpec((_G, 10), lambda i: (0, 0)),
        out_shape=jax.ShapeDtypeStruct((_G, 10), jnp.float32),
        scratch_shapes=[pltpu.VMEM((_G, _D), jnp.float32)],
    )(h, hmax, gid3, Wc, bc)


def kernel(x, edge_index, graph_ids, W0, b0, g0, be0, W1, b1, g1, be1,
           W2, b2, g2, be2, Wc, bc):
    pad = _E_PAD - _E
    pad_src = jnp.arange(pad, dtype=jnp.int32) * 997 % _N
    src2d = jnp.concatenate(
        [edge_index[0], pad_src]).reshape(_NW * _CPT, _CH)
    pad_dst = _DUMP + jnp.arange(pad, dtype=jnp.int32) % (_N_ACC - _N)
    dst2d = jnp.concatenate(
        [edge_index[1], pad_dst]).reshape(_NW * _CPT, _CH)
    zrows = jnp.zeros((_N_ACC, _D), jnp.float32)
    gid3 = graph_ids.reshape(_NB, 1, _R)
    bc2 = bc.reshape(1, 10)

    h = x
    for (W, b, gm, be) in ((W0, b0, g0, be0), (W1, b1, g1, be1)):
        a2 = _sc_scatter_sum(h, src2d, dst2d, zrows).reshape(_NC, _N_ACC, _D)
        z, st = _tc_linear_stats(a2, W, b.reshape(1, _D))
        h = _tc_norm(z, st, gm.reshape(1, _D), be.reshape(1, _D))

    a2 = _sc_scatter_sum(h, src2d, dst2d, zrows).reshape(_NC, _N_ACC, _D)
    z, st = _tc_linear_stats(a2, W2, b2.reshape(1, _D))
    h, hmax = _tc_norm3(z, st, g2.reshape(1, _D), be2.reshape(1, _D), Wc, bc2)

    return _tc_head(h, hmax, gid3, Wc, bc2)


# speedup vs baseline: 3.1307x; 1.0989x over previous
import functools

import jax
import jax.numpy as jnp
from jax import lax
from jax.experimental import pallas as pl
from jax.experimental.pallas import tpu as pltpu
from jax.experimental.pallas import tpu_sc as plsc

_N = 10000
_E = 320000
_D = 128
_G = 16

_NC = 2
_NS = 16
_NW = _NC * _NS
_CH = 64
_CPT = 160
_HC = 40
_NBUF = 4
_LG = 3
_EPT = _CPT * _CH
_E_PAD = _NW * _EPT
_N_ACC = 10112
_RPT = _N_ACC // _NS
_DUMP = _N

_R = 1000
_NB = _N // _R


def _sc_scatter_sum(h, src2d, dst2d, zrows):
    mesh = plsc.VectorSubcoreMesh(core_axis_name="c", subcore_axis_name="s")

    @functools.partial(
        pl.kernel,
        out_type=jax.ShapeDtypeStruct((_NC * _N_ACC, _D), jnp.float32),
        mesh=mesh,
        scratch_types=[
            pltpu.VMEM((_HC, _CH), jnp.int32),
            pltpu.VMEM((_HC, _CH), jnp.int32),
            pltpu.VMEM((_NBUF, _CH, _D), jnp.float32),
            pltpu.VMEM_SHARED((_N_ACC, _D), jnp.float32),
            pltpu.SemaphoreType.DMA,
            pltpu.SemaphoreType.DMA,
            pltpu.SemaphoreType.DMA,
            pltpu.SemaphoreType.DMA,
            pltpu.SemaphoreType.DMA,
            pltpu.SemaphoreType.DMA,
            pltpu.SemaphoreType.DMA,
            pltpu.SemaphoreType.DMA,
        ],
    )
    def k(h_hbm, src_hbm, dst_hbm, z_hbm, out_hbm,
          src_v, dst_v, rows_v, acc_sh,
          sem0, sem1, sem2, sem3, sem4, sem5, sem6, sem7):
        c = lax.axis_index("c")
        s = lax.axis_index("s")
        wid = c * _NS + s
        pltpu.sync_copy(z_hbm.at[pl.ds(s * _RPT, _RPT)],
                        acc_sh.at[pl.ds(s * _RPT, _RPT)])
        plsc.subcore_barrier()

        gsems = (sem0, sem1, sem2, sem3)
        ssems = (sem4, sem5, sem6, sem7)

        def gather(j, b):
            pltpu.async_copy(h_hbm.at[src_v.at[j]], rows_v.at[b], gsems[b])

        def gwait(j, b):
            pltpu.make_async_copy(
                h_hbm.at[src_v.at[j]], rows_v.at[b], gsems[b]).wait()

        def scat(j, b):
            pltpu.async_copy(rows_v.at[b], acc_sh.at[dst_v.at[j]],
                             ssems[b], add=True)

        def swait(j, b):
            pltpu.make_async_copy(
                rows_v.at[b], acc_sh.at[dst_v.at[j]], ssems[b]).wait()

        for half in range(_CPT // _HC):
            base = wid * _CPT + half * _HC
            pltpu.sync_copy(src_hbm.at[pl.ds(base, _HC)], src_v)
            pltpu.sync_copy(dst_hbm.at[pl.ds(base, _HC)], dst_v)

            def step(j, b):
                gwait(j, b)
                scat(j, b)
                nb = (b + _LG) % _NBUF
                swait(j + _LG - _NBUF, nb)
                gather(j + _LG, nb)

            for j in range(_LG):
                gather(j, j)
            for j in range(_NBUF):
                gwait(j, j)
                scat(j, j)
                if j + _LG < _NBUF:
                    gather(j + _LG, j + _LG)
                else:
                    swait(j + _LG - _NBUF, (j + _LG) % _NBUF)
                    gather(j + _LG, (j + _LG) % _NBUF)

            def body(g, carry):
                for b in range(_NBUF):
                    step(_NBUF * g + b, b)
                return carry

            lax.fori_loop(1, _HC // _NBUF - 1, body, 0)
            for j in range(_HC - _NBUF, _HC):
                gwait(j, j % _NBUF)
                scat(j, j % _NBUF)
                if j + _LG < _HC:
                    swait(j + _LG - _NBUF, (j + _LG) % _NBUF)
                    gather(j + _LG, (j + _LG) % _NBUF)
            for j in range(_HC - _NBUF, _HC):
                swait(j, j % _NBUF)

        plsc.subcore_barrier()
        row0 = c * _N_ACC + s * _RPT
        pltpu.sync_copy(acc_sh.at[pl.ds(s * _RPT, _RPT)],
                        out_hbm.at[pl.ds(row0, _RPT)])

    return k(h, src2d, dst2d, zrows)


def _dense_body(a0_ref, a1_ref, w_ref, b_ref, z_ref, st_ref, acc_ref):
    i = pl.program_id(0)
    a = a0_ref[0] + a1_ref[0]
    z = lax.dot_general(a, w_ref[...], (((1,), (1,)), ((), ())),
                        preferred_element_type=jnp.float32) + b_ref[...]
    z_ref[...] = z

    @pl.when(i == 0)
    def _init():
        acc_ref[...] = jnp.zeros_like(acc_ref)

    acc_ref[0:1, :] += jnp.sum(z, axis=0, keepdims=True)
    acc_ref[1:2, :] += jnp.sum(z * z, axis=0, keepdims=True)

    @pl.when(i == _NB - 1)
    def _fin():
        st_ref[...] = acc_ref[...]


def _tc_linear_stats(a2, W, b):
    return pl.pallas_call(
        _dense_body,
        grid=(_NB,),
        in_specs=[
            pl.BlockSpec((1, _R, _D), lambda i: (0, i, 0)),
            pl.BlockSpec((1, _R, _D), lambda i: (1, i, 0)),
            pl.BlockSpec((_D, _D), lambda i: (0, 0)),
            pl.BlockSpec((1, _D), lambda i: (0, 0)),
        ],
        out_specs=[
            pl.BlockSpec((_R, _D), lambda i: (i, 0)),
            pl.BlockSpec((2, _D), lambda i: (0, 0)),
        ],
        out_shape=[
            jax.ShapeDtypeStruct((_N, _D), jnp.float32),
            jax.ShapeDtypeStruct((2, _D), jnp.float32),
        ],
        scratch_shapes=[pltpu.VMEM((2, _D), jnp.float32)],
    )(a2, a2, W, b)


def _bn_elu(z, st_ref, g_ref, be_ref):
    mu = st_ref[0:1, :] * (1.0 / _N)
    var = st_ref[1:2, :] * (1.0 / _N) - mu * mu
    inv = lax.rsqrt(var + 1e-5)
    y = (z - mu) * (inv * g_ref[...]) + be_ref[...]
    return jnp.where(y > 0, y, jnp.exp(y) - 1.0)


def _norm_body(z_ref, st_ref, g_ref, be_ref, h_ref):
    h_ref[...] = _bn_elu(z_ref[...], st_ref, g_ref, be_ref)


def _tc_norm(z, st, g, be):
    return pl.pallas_call(
        _norm_body,
        grid=(_NB,),
        in_specs=[
            pl.BlockSpec((_R, _D), lambda i: (i, 0)),
            pl.BlockSpec((2, _D), lambda i: (0, 0)),
            pl.BlockSpec((1, _D), lambda i: (0, 0)),
            pl.BlockSpec((1, _D), lambda i: (0, 0)),
        ],
        out_specs=pl.BlockSpec((_R, _D), lambda i: (i, 0)),
        out_shape=jax.ShapeDtypeStruct((_N, _D), jnp.float32),
    )(z, st, g, be)


def _entropy(h, wc_ref, bc_ref):
    logits = lax.dot_general(h, wc_ref[...], (((1,), (1,)), ((), ())),
                             preferred_element_type=jnp.float32) + bc_ref[...]
    m = jnp.max(logits, axis=1, keepdims=True)
    lse = m + jnp.log(jnp.sum(jnp.exp(logits - m), axis=1, keepdims=True))
    logp = logits - lse
    return -jnp.sum(jnp.exp(logp) * logp, axis=1, keepdims=True)


def _norm3_body(z_ref, st_ref, g_ref, be_ref, wc_ref, bc_ref,
                h_ref, mx_ref, macc_ref):
    i = pl.program_id(0)
    h = _bn_elu(z_ref[...], st_ref, g_ref, be_ref)
    h_ref[...] = h
    hent = _entropy(h, wc_ref, bc_ref)
    bm = jnp.max(hent, axis=0, keepdims=True)

    @pl.when(i == 0)
    def _init():
        macc_ref[...] = jnp.full((1, 1), -jnp.inf, jnp.float32)

    macc_ref[...] = jnp.maximum(macc_ref[...], bm)

    @pl.when(i == _NB - 1)
    def _fin():
        mx_ref[...] = macc_ref[...]


def _tc_norm3(z, st, g, be, Wc, bc):
    return pl.pallas_call(
        _norm3_body,
        grid=(_NB,),
        in_specs=[
            pl.BlockSpec((_R, _D), lambda i: (i, 0)),
            pl.BlockSpec((2, _D), lambda i: (0, 0)),
            pl.BlockSpec((1, _D), lambda i: (0, 0)),
            pl.BlockSpec((1, _D), lambda i: (0, 0)),
            pl.BlockSpec((10, _D), lambda i: (0, 0)),
            pl.BlockSpec((1, 10), lambda i: (0, 0)),
        ],
        out_specs=[
            pl.BlockSpec((_R, _D), lambda i: (i, 0)),
            pl.BlockSpec((1, 1), lambda i: (0, 0)),
        ],
        out_shape=[
            jax.ShapeDtypeStruct((_N, _D), jnp.float32),
            jax.ShapeDtypeStruct((1, 1), jnp.float32),
        ],
        scratch_shapes=[pltpu.VMEM((1, 1), jnp.float32)],
    )(z, st, g, be, Wc, bc)


def _head_body(h_ref, mx_ref, gid_ref, wc_ref, bc_ref, t_ref, pool_ref):
    i = pl.program_id(0)
    h = h_ref[...]
    hent = _entropy(h, wc_ref, bc_ref)
    lam = 1.0 - hent / mx_ref[...]
    wgt = lam * h
    gid = gid_ref[0, 0, :]
    oh = (lax.broadcasted_iota(jnp.int32, (_G, _R), 0) == gid[None, :])
    part = lax.dot_general(oh.astype(jnp.float32), wgt,
                           (((1,), (0,)), ((), ())),
                           preferred_element_type=jnp.float32)

    @pl.when(i == 0)
    def _init():
        pool_ref[...] = jnp.zeros_like(pool_ref)

    pool_ref[...] += part

    @pl.when(i == _NB - 1)
    def _fin():
        t_ref[...] = lax.dot_general(
            pool_ref[...], wc_ref[...], (((1,), (1,)), ((), ())),
            preferred_element_type=jnp.float32) + bc_ref[...]


def _tc_head(h, hmax, gid3, Wc, bc):
    return pl.pallas_call(
        _head_body,
        grid=(_NB,),
        in_specs=[
            pl.BlockSpec((_R, _D), lambda i: (i, 0)),
            pl.BlockSpec((1, 1), lambda i: (0, 0)),
            pl.BlockSpec((1, 1, _R), lambda i: (i, 0, 0)),
            pl.BlockSpec((10, _D), lambda i: (0, 0)),
            pl.BlockSpec((1, 10), lambda i: (0, 0)),
        ],
        out_specs=pl.BlockSpec((_G, 10), lambda i: (0, 0)),
        out_shape=jax.ShapeDtypeStruct((_G, 10), jnp.float32),
        scratch_shapes=[pltpu.VMEM((_G, _D), jnp.float32)],
    )(h, hmax, gid3, Wc, bc)


def kernel(x, edge_index, graph_ids, W0, b0, g0, be0, W1, b1, g1, be1,
           W2, b2, g2, be2, Wc, bc):
    pad = _E_PAD - _E
    pad_src = jnp.arange(pad, dtype=jnp.int32) * 997 % _N
    src2d = jnp.concatenate(
        [edge_index[0], pad_src]).reshape(_NW * _CPT, _CH)
    pad_dst = _DUMP + jnp.arange(pad, dtype=jnp.int32) % (_N_ACC - _N)
    dst2d = jnp.concatenate(
        [edge_index[1], pad_dst]).reshape(_NW * _CPT, _CH)
    zrows = jnp.zeros((_N_ACC, _D), jnp.float32)
    gid3 = graph_ids.reshape(_NB, 1, _R)
    bc2 = bc.reshape(1, 10)

    h = x
    for (W, b, gm, be) in ((W0, b0, g0, be0), (W1, b1, g1, be1)):
        a2 = _sc_scatter_sum(h, src2d, dst2d, zrows).reshape(_NC, _N_ACC, _D)
        z, st = _tc_linear_stats(a2, W, b.reshape(1, _D))
        h = _tc_norm(z, st, gm.reshape(1, _D), be.reshape(1, _D))

    a2 = _sc_scatter_sum(h, src2d, dst2d, zrows).reshape(_NC, _N_ACC, _D)
    z, st = _tc_linear_stats(a2, W2, b2.reshape(1, _D))
    h, hmax = _tc_norm3(z, st, g2.reshape(1, _D), be2.reshape(1, _D), Wc, bc2)

    return _tc_head(h, hmax, gid3, Wc, bc2)


# speedup vs baseline: 3.2256x; 1.0303x over previous
import functools

import jax
import jax.numpy as jnp
from jax import lax
from jax.experimental import pallas as pl
from jax.experimental.pallas import tpu as pltpu
from jax.experimental.pallas import tpu_sc as plsc

_N = 10000
_E = 320000
_D = 128
_G = 16

_NC = 2
_NS = 16
_NW = _NC * _NS
_CH = 64
_CPT = 160
_HC = 40
_NBUF = 4
_LG = 3
_EPT = _CPT * _CH
_E_PAD = _NW * _EPT
_N_ACC = 10112
_RPT = _N_ACC // _NS
_DUMP = _N

_R = 1000
_NB = _N // _R


def _sc_scatter_sum(h, src2d, dst2d, zrows):
    mesh = plsc.VectorSubcoreMesh(core_axis_name="c", subcore_axis_name="s")

    @functools.partial(
        pl.kernel,
        out_type=jax.ShapeDtypeStruct((_NC * _N_ACC, _D), jnp.float32),
        mesh=mesh,
        scratch_types=[
            pltpu.VMEM((_HC, _CH), jnp.int32),
            pltpu.VMEM((_HC, _CH), jnp.int32),
            pltpu.VMEM((_NBUF, _CH, _D), jnp.float32),
            pltpu.VMEM_SHARED((_N_ACC, _D), jnp.float32),
            pltpu.SemaphoreType.DMA,
            pltpu.SemaphoreType.DMA,
            pltpu.SemaphoreType.DMA,
            pltpu.SemaphoreType.DMA,
            pltpu.SemaphoreType.DMA,
            pltpu.SemaphoreType.DMA,
            pltpu.SemaphoreType.DMA,
            pltpu.SemaphoreType.DMA,
        ],
    )
    def k(h_hbm, src_hbm, dst_hbm, z_hbm, out_hbm,
          src_v, dst_v, rows_v, acc_sh,
          sem0, sem1, sem2, sem3, sem4, sem5, sem6, sem7):
        c = lax.axis_index("c")
        s = lax.axis_index("s")
        wid = c * _NS + s
        pltpu.sync_copy(z_hbm.at[pl.ds(s * _RPT, _RPT)],
                        acc_sh.at[pl.ds(s * _RPT, _RPT)])
        plsc.subcore_barrier()

        gsems = (sem0, sem1, sem2, sem3)
        ssems = (sem4, sem5, sem6, sem7)

        def gather(j, b):
            pltpu.async_copy(h_hbm.at[src_v.at[j]], rows_v.at[b], gsems[b])

        def gwait(j, b):
            pltpu.make_async_copy(
                h_hbm.at[src_v.at[j]], rows_v.at[b], gsems[b]).wait()

        def scat(j, b):
            pltpu.async_copy(rows_v.at[b], acc_sh.at[dst_v.at[j]],
                             ssems[b], add=True)

        def swait(j, b):
            pltpu.make_async_copy(
                rows_v.at[b], acc_sh.at[dst_v.at[j]], ssems[b]).wait()

        for half in range(_CPT // _HC):
            base = wid * _CPT + half * _HC
            pltpu.sync_copy(src_hbm.at[pl.ds(base, _HC)], src_v)
            pltpu.sync_copy(dst_hbm.at[pl.ds(base, _HC)], dst_v)

            def step(j, b):
                gwait(j, b)
                scat(j, b)
                nb = (b + _LG) % _NBUF
                swait(j + _LG - _NBUF, nb)
                gather(j + _LG, nb)

            for j in range(_LG):
                gather(j, j)
            for j in range(_NBUF):
                gwait(j, j)
                scat(j, j)
                if j + _LG < _NBUF:
                    gather(j + _LG, j + _LG)
                else:
                    swait(j + _LG - _NBUF, (j + _LG) % _NBUF)
                    gather(j + _LG, (j + _LG) % _NBUF)

            def body(g, carry):
                for b in range(_NBUF):
                    step(_NBUF * g + b, b)
                return carry

            lax.fori_loop(1, _HC // _NBUF - 1, body, 0)
            for j in range(_HC - _NBUF, _HC):
                gwait(j, j % _NBUF)
                scat(j, j % _NBUF)
                if j + _LG < _HC:
                    swait(j + _LG - _NBUF, (j + _LG) % _NBUF)
                    gather(j + _LG, (j + _LG) % _NBUF)
            for j in range(_HC - _NBUF, _HC):
                swait(j, j % _NBUF)

        plsc.subcore_barrier()
        row0 = c * _N_ACC + s * _RPT
        pltpu.sync_copy(acc_sh.at[pl.ds(s * _RPT, _RPT)],
                        out_hbm.at[pl.ds(row0, _RPT)])

    return k(h, src2d, dst2d, zrows)


def _bn_elu(z, acc_ref, g_ref, be_ref):
    mu = acc_ref[0:1, :] * (1.0 / _N)
    var = acc_ref[1:2, :] * (1.0 / _N) - mu * mu
    inv = lax.rsqrt(var + 1e-5)
    y = (z - mu) * (inv * g_ref[...]) + be_ref[...]
    return jnp.where(y > 0, y, jnp.exp(y) - 1.0)


def _layer_body(a0_ref, a1_ref, w_ref, b_ref, g_ref, be_ref,
                h_ref, z_sc, acc_ref):
    p = pl.program_id(0)
    i = pl.program_id(1)

    @pl.when(p == 0)
    def _compute():
        a = a0_ref[0] + a1_ref[0]
        z = lax.dot_general(a, w_ref[...], (((1,), (1,)), ((), ())),
                            preferred_element_type=jnp.float32) + b_ref[...]
        z_sc[pl.ds(i * _R, _R), :] = z

        @pl.when(i == 0)
        def _init():
            acc_ref[...] = jnp.zeros_like(acc_ref)

        acc_ref[0:1, :] += jnp.sum(z, axis=0, keepdims=True)
        acc_ref[1:2, :] += jnp.sum(z * z, axis=0, keepdims=True)

    @pl.when(p == 1)
    def _norm():
        z = z_sc[pl.ds(i * _R, _R), :]
        h_ref[...] = _bn_elu(z, acc_ref, g_ref, be_ref)


def _tc_layer(a2, W, b, g, be):
    return pl.pallas_call(
        _layer_body,
        grid=(2, _NB),
        in_specs=[
            pl.BlockSpec((1, _R, _D), lambda p, i: (0, i * (1 - p), 0)),
            pl.BlockSpec((1, _R, _D), lambda p, i: (1, i * (1 - p), 0)),
            pl.BlockSpec((_D, _D), lambda p, i: (0, 0)),
            pl.BlockSpec((1, _D), lambda p, i: (0, 0)),
            pl.BlockSpec((1, _D), lambda p, i: (0, 0)),
            pl.BlockSpec((1, _D), lambda p, i: (0, 0)),
        ],
        out_specs=pl.BlockSpec((_R, _D), lambda p, i: (i * p, 0)),
        out_shape=jax.ShapeDtypeStruct((_N, _D), jnp.float32),
        scratch_shapes=[
            pltpu.VMEM((_N, _D), jnp.float32),
            pltpu.VMEM((2, _D), jnp.float32),
        ],
    )(a2, a2, W, b, g, be)


def _entropy(h, wc_ref, bc_ref):
    logits = lax.dot_general(h, wc_ref[...], (((1,), (1,)), ((), ())),
                             preferred_element_type=jnp.float32) + bc_ref[...]
    m = jnp.max(logits, axis=1, keepdims=True)
    lse = m + jnp.log(jnp.sum(jnp.exp(logits - m), axis=1, keepdims=True))
    logp = logits - lse
    return -jnp.sum(jnp.exp(logp) * logp, axis=1, keepdims=True)


def _layer3_body(a0_ref, a1_ref, w_ref, b_ref, g_ref, be_ref, wc_ref, bc_ref,
                 h_ref, mx_ref, z_sc, acc_ref, macc_ref):
    p = pl.program_id(0)
    i = pl.program_id(1)

    @pl.when(p == 0)
    def _compute():
        a = a0_ref[0] + a1_ref[0]
        z = lax.dot_general(a, w_ref[...], (((1,), (1,)), ((), ())),
                            preferred_element_type=jnp.float32) + b_ref[...]
        z_sc[pl.ds(i * _R, _R), :] = z

        @pl.when(i == 0)
        def _init():
            acc_ref[...] = jnp.zeros_like(acc_ref)
            macc_ref[...] = jnp.full((1, 1), -jnp.inf, jnp.float32)

        acc_ref[0:1, :] += jnp.sum(z, axis=0, keepdims=True)
        acc_ref[1:2, :] += jnp.sum(z * z, axis=0, keepdims=True)

    @pl.when(p == 1)
    def _norm():
        z = z_sc[pl.ds(i * _R, _R), :]
        h = _bn_elu(z, acc_ref, g_ref, be_ref)
        h_ref[...] = h
        hent = _entropy(h, wc_ref, bc_ref)
        macc_ref[...] = jnp.maximum(macc_ref[...],
                                    jnp.max(hent, axis=0, keepdims=True))

        @pl.when(i == _NB - 1)
        def _fin():
            mx_ref[...] = macc_ref[...]


def _tc_layer3(a2, W, b, g, be, Wc, bc):
    return pl.pallas_call(
        _layer3_body,
        grid=(2, _NB),
        in_specs=[
            pl.BlockSpec((1, _R, _D), lambda p, i: (0, i * (1 - p), 0)),
            pl.BlockSpec((1, _R, _D), lambda p, i: (1, i * (1 - p), 0)),
            pl.BlockSpec((_D, _D), lambda p, i: (0, 0)),
            pl.BlockSpec((1, _D), lambda p, i: (0, 0)),
            pl.BlockSpec((1, _D), lambda p, i: (0, 0)),
            pl.BlockSpec((1, _D), lambda p, i: (0, 0)),
            pl.BlockSpec((10, _D), lambda p, i: (0, 0)),
            pl.BlockSpec((1, 10), lambda p, i: (0, 0)),
        ],
        out_specs=[
            pl.BlockSpec((_R, _D), lambda p, i: (i * p, 0)),
            pl.BlockSpec((1, 1), lambda p, i: (0, 0)),
        ],
        out_shape=[
            jax.ShapeDtypeStruct((_N, _D), jnp.float32),
            jax.ShapeDtypeStruct((1, 1), jnp.float32),
        ],
        scratch_shapes=[
            pltpu.VMEM((_N, _D), jnp.float32),
            pltpu.VMEM((2, _D), jnp.float32),
            pltpu.VMEM((1, 1), jnp.float32),
        ],
    )(a2, a2, W, b, g, be, Wc, bc)


def _head_body(h_ref, mx_ref, gid_ref, wc_ref, bc_ref, t_ref, pool_ref):
    i = pl.program_id(0)
    h = h_ref[...]
    hent = _entropy(h, wc_ref, bc_ref)
    lam = 1.0 - hent / mx_ref[...]
    wgt = lam * h
    gid = gid_ref[0, 0, :]
    oh = (lax.broadcasted_iota(jnp.int32, (_G, _R), 0) == gid[None, :])
    part = lax.dot_general(oh.astype(jnp.float32), wgt,
                           (((1,), (0,)), ((), ())),
                           preferred_element_type=jnp.float32)

    @pl.when(i == 0)
    def _init():
        pool_ref[...] = jnp.zeros_like(pool_ref)

    pool_ref[...] += part

    @pl.when(i == _NB - 1)
    def _fin():
        t_ref[...] = lax.dot_general(
            pool_ref[...], wc_ref[...], (((1,), (1,)), ((), ())),
            preferred_element_type=jnp.float32) + bc_ref[...]


def _tc_head(h, hmax, gid3, Wc, bc):
    return pl.pallas_call(
        _head_body,
        grid=(_NB,),
        in_specs=[
            pl.BlockSpec((_R, _D), lambda i: (i, 0)),
            pl.BlockSpec((1, 1), lambda i: (0, 0)),
            pl.BlockSpec((1, 1, _R), lambda i: (i, 0, 0)),
            pl.BlockSpec((10, _D), lambda i: (0, 0)),
            pl.BlockSpec((1, 10), lambda i: (0, 0)),
        ],
        out_specs=pl.BlockSpec((_G, 10), lambda i: (0, 0)),
        out_shape=jax.ShapeDtypeStruct((_G, 10), jnp.float32),
        scratch_shapes=[pltpu.VMEM((_G, _D), jnp.float32)],
    )(h, hmax, gid3, Wc, bc)


def kernel(x, edge_index, graph_ids, W0, b0, g0, be0, W1, b1, g1, be1,
           W2, b2, g2, be2, Wc, bc):
    pad = _E_PAD - _E
    pad_src = jnp.arange(pad, dtype=jnp.int32) * 997 % _N
    src2d = jnp.concatenate(
        [edge_index[0], pad_src]).reshape(_NW * _CPT, _CH)
    pad_dst = _DUMP + jnp.arange(pad, dtype=jnp.int32) % (_N_ACC - _N)
    dst2d = jnp.concatenate(
        [edge_index[1], pad_dst]).reshape(_NW * _CPT, _CH)
    zrows = jnp.zeros((_N_ACC, _D), jnp.float32)
    gid3 = graph_ids.reshape(_NB, 1, _R)
    bc2 = bc.reshape(1, 10)

    h = x
    for (W, b, gm, be) in ((W0, b0, g0, be0), (W1, b1, g1, be1)):
        a2 = _sc_scatter_sum(h, src2d, dst2d, zrows).reshape(_NC, _N_ACC, _D)
        h = _tc_layer(a2, W, b.reshape(1, _D),
                      gm.reshape(1, _D), be.reshape(1, _D))

    a2 = _sc_scatter_sum(h, src2d, dst2d, zrows).reshape(_NC, _N_ACC, _D)
    h, hmax = _tc_layer3(a2, W2, b2.reshape(1, _D),
                         g2.reshape(1, _D), be2.reshape(1, _D), Wc, bc2)

    return _tc_head(h, hmax, gid3, Wc, bc2)


# speedup vs baseline: 3.2457x; 1.0062x over previous
import functools

import jax
import jax.numpy as jnp
from jax import lax
from jax.experimental import pallas as pl
from jax.experimental.pallas import tpu as pltpu
from jax.experimental.pallas import tpu_sc as plsc

_N = 10000
_E = 320000
_D = 128
_G = 16

_NC = 2
_NS = 16
_NW = _NC * _NS
_CH = 64
_CPT = 160
_HC = 40
_NBUF = 4
_LG = 3
_EPT = _CPT * _CH
_E_PAD = _NW * _EPT
_N_ACC = 10112
_RPT = _N_ACC // _NS
_DUMP = _N

_R = 1000
_NB = _N // _R


def _sc_scatter_sum(h, src2d, dst2d, zrows):
    mesh = plsc.VectorSubcoreMesh(core_axis_name="c", subcore_axis_name="s")

    @functools.partial(
        pl.kernel,
        out_type=jax.ShapeDtypeStruct((_NC * _N_ACC, _D), jnp.float32),
        mesh=mesh,
        scratch_types=[
            pltpu.VMEM((_HC, _CH), jnp.int32),
            pltpu.VMEM((_HC, _CH), jnp.int32),
            pltpu.VMEM((_NBUF, _CH, _D), jnp.float32),
            pltpu.VMEM_SHARED((_N_ACC, _D), jnp.float32),
            pltpu.SemaphoreType.DMA,
            pltpu.SemaphoreType.DMA,
            pltpu.SemaphoreType.DMA,
            pltpu.SemaphoreType.DMA,
            pltpu.SemaphoreType.DMA,
            pltpu.SemaphoreType.DMA,
            pltpu.SemaphoreType.DMA,
            pltpu.SemaphoreType.DMA,
        ],
    )
    def k(h_hbm, src_hbm, dst_hbm, z_hbm, out_hbm,
          src_v, dst_v, rows_v, acc_sh,
          sem0, sem1, sem2, sem3, sem4, sem5, sem6, sem7):
        c = lax.axis_index("c")
        s = lax.axis_index("s")
        wid = c * _NS + s
        pltpu.sync_copy(z_hbm.at[pl.ds(s * _RPT, _RPT)],
                        acc_sh.at[pl.ds(s * _RPT, _RPT)])
        plsc.subcore_barrier()

        gsems = (sem0, sem1, sem2, sem3)
        ssems = (sem4, sem5, sem6, sem7)

        def gather(j, b):
            pltpu.async_copy(h_hbm.at[src_v.at[j]], rows_v.at[b], gsems[b])

        def gwait(j, b):
            pltpu.make_async_copy(
                h_hbm.at[src_v.at[j]], rows_v.at[b], gsems[b]).wait()

        def scat(j, b):
            pltpu.async_copy(rows_v.at[b], acc_sh.at[dst_v.at[j]],
                             ssems[b], add=True)

        def swait(j, b):
            pltpu.make_async_copy(
                rows_v.at[b], acc_sh.at[dst_v.at[j]], ssems[b]).wait()

        for half in range(_CPT // _HC):
            base = wid * _CPT + half * _HC
            pltpu.sync_copy(src_hbm.at[pl.ds(base, _HC)], src_v)
            pltpu.sync_copy(dst_hbm.at[pl.ds(base, _HC)], dst_v)

            def step(j, b):
                gwait(j, b)
                scat(j, b)
                nb = (b + _LG) % _NBUF
                swait(j + _LG - _NBUF, nb)
                gather(j + _LG, nb)

            for j in range(_LG):
                gather(j, j)
            for j in range(_NBUF):
                gwait(j, j)
                scat(j, j)
                if j + _LG < _NBUF:
                    gather(j + _LG, j + _LG)
                else:
                    swait(j + _LG - _NBUF, (j + _LG) % _NBUF)
                    gather(j + _LG, (j + _LG) % _NBUF)

            def body(g, carry):
                for b in range(_NBUF):
                    step(_NBUF * g + b, b)
                return carry

            lax.fori_loop(1, _HC // _NBUF - 1, body, 0)
            for j in range(_HC - _NBUF, _HC):
                gwait(j, j % _NBUF)
                scat(j, j % _NBUF)
                if j + _LG < _HC:
                    swait(j + _LG - _NBUF, (j + _LG) % _NBUF)
                    gather(j + _LG, (j + _LG) % _NBUF)
            for j in range(_HC - _NBUF, _HC):
                swait(j, j % _NBUF)

        plsc.subcore_barrier()
        row0 = c * _N_ACC + s * _RPT
        pltpu.sync_copy(acc_sh.at[pl.ds(s * _RPT, _RPT)],
                        out_hbm.at[pl.ds(row0, _RPT)])

    return k(h, src2d, dst2d, zrows)


def _bn_elu(z, acc_ref, g_ref, be_ref):
    mu = acc_ref[0:1, :] * (1.0 / _N)
    var = acc_ref[1:2, :] * (1.0 / _N) - mu * mu
    inv = lax.rsqrt(var + 1e-5)
    y = (z - mu) * (inv * g_ref[...]) + be_ref[...]
    return jnp.where(y > 0, y, jnp.exp(y) - 1.0)


def _layer_body(a0_ref, a1_ref, w_ref, b_ref, g_ref, be_ref,
                h_ref, z_sc, acc_ref):
    p = pl.program_id(0)
    i = pl.program_id(1)

    @pl.when(p == 0)
    def _compute():
        a = a0_ref[0] + a1_ref[0]
        z = lax.dot_general(a, w_ref[...], (((1,), (1,)), ((), ())),
                            preferred_element_type=jnp.float32) + b_ref[...]
        z_sc[pl.ds(i * _R, _R), :] = z

        @pl.when(i == 0)
        def _init():
            acc_ref[...] = jnp.zeros_like(acc_ref)

        acc_ref[0:1, :] += jnp.sum(z, axis=0, keepdims=True)
        acc_ref[1:2, :] += jnp.sum(z * z, axis=0, keepdims=True)

    @pl.when(p == 1)
    def _norm():
        z = z_sc[pl.ds(i * _R, _R), :]
        h_ref[...] = _bn_elu(z, acc_ref, g_ref, be_ref)


def _tc_layer(a2, W, b, g, be):
    return pl.pallas_call(
        _layer_body,
        grid=(2, _NB),
        in_specs=[
            pl.BlockSpec((1, _R, _D), lambda p, i: (0, i * (1 - p), 0)),
            pl.BlockSpec((1, _R, _D), lambda p, i: (1, i * (1 - p), 0)),
            pl.BlockSpec((_D, _D), lambda p, i: (0, 0)),
            pl.BlockSpec((1, _D), lambda p, i: (0, 0)),
            pl.BlockSpec((1, _D), lambda p, i: (0, 0)),
            pl.BlockSpec((1, _D), lambda p, i: (0, 0)),
        ],
        out_specs=pl.BlockSpec((_R, _D), lambda p, i: (i * p, 0)),
        out_shape=jax.ShapeDtypeStruct((_N, _D), jnp.float32),
        scratch_shapes=[
            pltpu.VMEM((_N, _D), jnp.float32),
            pltpu.VMEM((2, _D), jnp.float32),
        ],
    )(a2, a2, W, b, g, be)


def _entropy(h, wc_ref, bc_ref):
    logits = lax.dot_general(h, wc_ref[...], (((1,), (1,)), ((), ())),
                             preferred_element_type=jnp.float32) + bc_ref[...]
    m = jnp.max(logits, axis=1, keepdims=True)
    lse = m + jnp.log(jnp.sum(jnp.exp(logits - m), axis=1, keepdims=True))
    logp = logits - lse
    return -jnp.sum(jnp.exp(logp) * logp, axis=1, keepdims=True)


def _final_body(a0_ref, a1_ref, w_ref, b_ref, g_ref, be_ref, wc_ref, bc_ref,
                gid_ref, t_ref, z_sc, acc_ref, macc_ref, pool_ref):
    p = pl.program_id(0)
    i = pl.program_id(1)

    @pl.when(p == 0)
    def _compute():
        a = a0_ref[0] + a1_ref[0]
        z = lax.dot_general(a, w_ref[...], (((1,), (1,)), ((), ())),
                            preferred_element_type=jnp.float32) + b_ref[...]
        z_sc[pl.ds(i * _R, _R), :] = z

        @pl.when(i == 0)
        def _init():
            acc_ref[...] = jnp.zeros_like(acc_ref)
            macc_ref[...] = jnp.full((1, 1), -jnp.inf, jnp.float32)
            pool_ref[...] = jnp.zeros_like(pool_ref)

        acc_ref[0:1, :] += jnp.sum(z, axis=0, keepdims=True)
        acc_ref[1:2, :] += jnp.sum(z * z, axis=0, keepdims=True)

    @pl.when(p == 1)
    def _norm():
        z = z_sc[pl.ds(i * _R, _R), :]
        h = _bn_elu(z, acc_ref, g_ref, be_ref)
        z_sc[pl.ds(i * _R, _R), :] = h
        hent = _entropy(h, wc_ref, bc_ref)
        macc_ref[...] = jnp.maximum(macc_ref[...],
                                    jnp.max(hent, axis=0, keepdims=True))

    @pl.when(p == 2)
    def _head():
        h = z_sc[pl.ds(i * _R, _R), :]
        hent = _entropy(h, wc_ref, bc_ref)
        lam = 1.0 - hent / macc_ref[...]
        wgt = lam * h
        gid = gid_ref[0, 0, :]
        oh = (lax.broadcasted_iota(jnp.int32, (_G, _R), 0) == gid[None, :])
        pool_ref[...] += lax.dot_general(
            oh.astype(jnp.float32), wgt, (((1,), (0,)), ((), ())),
            preferred_element_type=jnp.float32)

        @pl.when(i == _NB - 1)
        def _fin():
            t_ref[...] = lax.dot_general(
                pool_ref[...], wc_ref[...], (((1,), (1,)), ((), ())),
                preferred_element_type=jnp.float32) + bc_ref[...]


def _tc_final(a2, W, b, g, be, Wc, bc, gid3):
    return pl.pallas_call(
        _final_body,
        grid=(3, _NB),
        in_specs=[
            pl.BlockSpec((1, _R, _D),
                         lambda p, i: (0, i * ((1 - p) * (2 - p) // 2), 0)),
            pl.BlockSpec((1, _R, _D),
                         lambda p, i: (1, i * ((1 - p) * (2 - p) // 2), 0)),
            pl.BlockSpec((_D, _D), lambda p, i: (0, 0)),
            pl.BlockSpec((1, _D), lambda p, i: (0, 0)),
            pl.BlockSpec((1, _D), lambda p, i: (0, 0)),
            pl.BlockSpec((1, _D), lambda p, i: (0, 0)),
            pl.BlockSpec((10, _D), lambda p, i: (0, 0)),
            pl.BlockSpec((1, 10), lambda p, i: (0, 0)),
            pl.BlockSpec((1, 1, _R), lambda p, i: (i * (p // 2), 0, 0)),
        ],
        out_specs=pl.BlockSpec((_G, 10), lambda p, i: (0, 0)),
        out_shape=jax.ShapeDtypeStruct((_G, 10), jnp.float32),
        scratch_shapes=[
            pltpu.VMEM((_N, _D), jnp.float32),
            pltpu.VMEM((2, _D), jnp.float32),
            pltpu.VMEM((1, 1), jnp.float32),
            pltpu.VMEM((_G, _D), jnp.float32),
        ],
    )(a2, a2, W, b, g, be, Wc, bc, gid3)


def kernel(x, edge_index, graph_ids, W0, b0, g0, be0, W1, b1, g1, be1,
           W2, b2, g2, be2, Wc, bc):
    pad = _E_PAD - _E
    pad_src = jnp.arange(pad, dtype=jnp.int32) * 997 % _N
    src2d = jnp.concatenate(
        [edge_index[0], pad_src]).reshape(_NW * _CPT, _CH)
    pad_dst = _DUMP + jnp.arange(pad, dtype=jnp.int32) % (_N_ACC - _N)
    dst2d = jnp.concatenate(
        [edge_index[1], pad_dst]).reshape(_NW * _CPT, _CH)
    zrows = jnp.zeros((_N_ACC, _D), jnp.float32)
    gid3 = graph_ids.reshape(_NB, 1, _R)
    bc2 = bc.reshape(1, 10)

    h = x
    for (W, b, gm, be) in ((W0, b0, g0, be0), (W1, b1, g1, be1)):
        a2 = _sc_scatter_sum(h, src2d, dst2d, zrows).reshape(_NC, _N_ACC, _D)
        h = _tc_layer(a2, W, b.reshape(1, _D),
                      gm.reshape(1, _D), be.reshape(1, _D))

    a2 = _sc_scatter_sum(h, src2d, dst2d, zrows).reshape(_NC, _N_ACC, _D)
    return _tc_final(a2, W2, b2.reshape(1, _D),
                     g2.reshape(1, _D), be2.reshape(1, _D), Wc, bc2, gid3)


# speedup vs baseline: 3.4132x; 1.0516x over previous
import functools

import jax
import jax.numpy as jnp
from jax import lax
from jax.experimental import pallas as pl
from jax.experimental.pallas import tpu as pltpu
from jax.experimental.pallas import tpu_sc as plsc

_N = 10000
_E = 320000
_D = 128
_G = 16

_NC = 2
_NS = 16
_NW = _NC * _NS
_CH = 64
_CPT = 160
_HC = 40
_NBUF = 4
_LG = 3
_EPT = _CPT * _CH
_E_PAD = _NW * _EPT
_N_ACC = 10112
_RPT = _N_ACC // _NS
_DUMP = _N

_R = 2000
_NB = _N // _R


def _sc_scatter_sum(h, src2d, dst2d, zrows):
    mesh = plsc.VectorSubcoreMesh(core_axis_name="c", subcore_axis_name="s")

    @functools.partial(
        pl.kernel,
        out_type=jax.ShapeDtypeStruct((_NC * _N_ACC, _D), jnp.float32),
        mesh=mesh,
        scratch_types=[
            pltpu.VMEM((_HC, _CH), jnp.int32),
            pltpu.VMEM((_HC, _CH), jnp.int32),
            pltpu.VMEM((_NBUF, _CH, _D), jnp.float32),
            pltpu.VMEM_SHARED((_N_ACC, _D), jnp.float32),
            pltpu.SemaphoreType.DMA,
            pltpu.SemaphoreType.DMA,
            pltpu.SemaphoreType.DMA,
            pltpu.SemaphoreType.DMA,
            pltpu.SemaphoreType.DMA,
            pltpu.SemaphoreType.DMA,
            pltpu.SemaphoreType.DMA,
            pltpu.SemaphoreType.DMA,
        ],
    )
    def k(h_hbm, src_hbm, dst_hbm, z_hbm, out_hbm,
          src_v, dst_v, rows_v, acc_sh,
          sem0, sem1, sem2, sem3, sem4, sem5, sem6, sem7):
        c = lax.axis_index("c")
        s = lax.axis_index("s")
        wid = c * _NS + s
        pltpu.sync_copy(z_hbm.at[pl.ds(s * _RPT, _RPT)],
                        acc_sh.at[pl.ds(s * _RPT, _RPT)])
        plsc.subcore_barrier()

        gsems = (sem0, sem1, sem2, sem3)
        ssems = (sem4, sem5, sem6, sem7)

        def gather(j, b):
            pltpu.async_copy(h_hbm.at[src_v.at[j]], rows_v.at[b], gsems[b])

        def gwait(j, b):
            pltpu.make_async_copy(
                h_hbm.at[src_v.at[j]], rows_v.at[b], gsems[b]).wait()

        def scat(j, b):
            pltpu.async_copy(rows_v.at[b], acc_sh.at[dst_v.at[j]],
                             ssems[b], add=True)

        def swait(j, b):
            pltpu.make_async_copy(
                rows_v.at[b], acc_sh.at[dst_v.at[j]], ssems[b]).wait()

        for half in range(_CPT // _HC):
            base = wid * _CPT + half * _HC
            pltpu.sync_copy(src_hbm.at[pl.ds(base, _HC)], src_v)
            pltpu.sync_copy(dst_hbm.at[pl.ds(base, _HC)], dst_v)

            def step(j, b):
                gwait(j, b)
                scat(j, b)
                nb = (b + _LG) % _NBUF
                swait(j + _LG - _NBUF, nb)
                gather(j + _LG, nb)

            for j in range(_LG):
                gather(j, j)
            for j in range(_NBUF):
                gwait(j, j)
                scat(j, j)
                if j + _LG < _NBUF:
                    gather(j + _LG, j + _LG)
                else:
                    swait(j + _LG - _NBUF, (j + _LG) % _NBUF)
                    gather(j + _LG, (j + _LG) % _NBUF)

            def body(g, carry):
                for b in range(_NBUF):
                    step(_NBUF * g + b, b)
                return carry

            lax.fori_loop(1, _HC // _NBUF - 1, body, 0)
            for j in range(_HC - _NBUF, _HC):
                gwait(j, j % _NBUF)
                scat(j, j % _NBUF)
                if j + _LG < _HC:
                    swait(j + _LG - _NBUF, (j + _LG) % _NBUF)
                    gather(j + _LG, (j + _LG) % _NBUF)
            for j in range(_HC - _NBUF, _HC):
                swait(j, j % _NBUF)

        plsc.subcore_barrier()
        row0 = c * _N_ACC + s * _RPT
        pltpu.sync_copy(acc_sh.at[pl.ds(s * _RPT, _RPT)],
                        out_hbm.at[pl.ds(row0, _RPT)])

    return k(h, src2d, dst2d, zrows)


def _bn_elu(z, acc_ref, g_ref, be_ref):
    mu = acc_ref[0:1, :] * (1.0 / _N)
    var = acc_ref[1:2, :] * (1.0 / _N) - mu * mu
    inv = lax.rsqrt(var + 1e-5)
    y = (z - mu) * (inv * g_ref[...]) + be_ref[...]
    return jnp.where(y > 0, y, jnp.exp(y) - 1.0)


def _layer_body(a0_ref, a1_ref, w_ref, b_ref, g_ref, be_ref,
                h_ref, z_sc, acc_ref):
    p = pl.program_id(0)
    i = pl.program_id(1)

    @pl.when(p == 0)
    def _compute():
        a = a0_ref[0] + a1_ref[0]
        z = lax.dot_general(a, w_ref[...], (((1,), (1,)), ((), ())),
                            preferred_element_type=jnp.float32) + b_ref[...]
        z_sc[pl.ds(i * _R, _R), :] = z

        @pl.when(i == 0)
        def _init():
            acc_ref[...] = jnp.zeros_like(acc_ref)

        acc_ref[0:1, :] += jnp.sum(z, axis=0, keepdims=True)
        acc_ref[1:2, :] += jnp.sum(z * z, axis=0, keepdims=True)

    @pl.when(p == 1)
    def _norm():
        z = z_sc[pl.ds(i * _R, _R), :]
        h_ref[...] = _bn_elu(z, acc_ref, g_ref, be_ref)


def _tc_layer(a2, W, b, g, be):
    return pl.pallas_call(
        _layer_body,
        grid=(2, _NB),
        in_specs=[
            pl.BlockSpec((1, _R, _D), lambda p, i: (0, i * (1 - p), 0)),
            pl.BlockSpec((1, _R, _D), lambda p, i: (1, i * (1 - p), 0)),
            pl.BlockSpec((_D, _D), lambda p, i: (0, 0)),
            pl.BlockSpec((1, _D), lambda p, i: (0, 0)),
            pl.BlockSpec((1, _D), lambda p, i: (0, 0)),
            pl.BlockSpec((1, _D), lambda p, i: (0, 0)),
        ],
        out_specs=pl.BlockSpec((_R, _D), lambda p, i: (i * p, 0)),
        out_shape=jax.ShapeDtypeStruct((_N, _D), jnp.float32),
        scratch_shapes=[
            pltpu.VMEM((_N, _D), jnp.float32),
            pltpu.VMEM((2, _D), jnp.float32),
        ],
    )(a2, a2, W, b, g, be)


def _entropy(h, wc_ref, bc_ref):
    logits = lax.dot_general(h, wc_ref[...], (((1,), (1,)), ((), ())),
                             preferred_element_type=jnp.float32) + bc_ref[...]
    m = jnp.max(logits, axis=1, keepdims=True)
    lse = m + jnp.log(jnp.sum(jnp.exp(logits - m), axis=1, keepdims=True))
    logp = logits - lse
    return -jnp.sum(jnp.exp(logp) * logp, axis=1, keepdims=True)


def _final_body(a0_ref, a1_ref, w_ref, b_ref, g_ref, be_ref, wc_ref, bc_ref,
                gid_ref, t_ref, z_sc, acc_ref, macc_ref, pool_ref):
    p = pl.program_id(0)
    i = pl.program_id(1)

    @pl.when(p == 0)
    def _compute():
        a = a0_ref[0] + a1_ref[0]
        z = lax.dot_general(a, w_ref[...], (((1,), (1,)), ((), ())),
                            preferred_element_type=jnp.float32) + b_ref[...]
        z_sc[pl.ds(i * _R, _R), :] = z

        @pl.when(i == 0)
        def _init():
            acc_ref[...] = jnp.zeros_like(acc_ref)
            macc_ref[...] = jnp.full((1, 1), -jnp.inf, jnp.float32)
            pool_ref[...] = jnp.zeros_like(pool_ref)

        acc_ref[0:1, :] += jnp.sum(z, axis=0, keepdims=True)
        acc_ref[1:2, :] += jnp.sum(z * z, axis=0, keepdims=True)

    @pl.when(p == 1)
    def _norm():
        z = z_sc[pl.ds(i * _R, _R), :]
        h = _bn_elu(z, acc_ref, g_ref, be_ref)
        z_sc[pl.ds(i * _R, _R), :] = h
        hent = _entropy(h, wc_ref, bc_ref)
        macc_ref[...] = jnp.maximum(macc_ref[...],
                                    jnp.max(hent, axis=0, keepdims=True))

    @pl.when(p == 2)
    def _head():
        h = z_sc[pl.ds(i * _R, _R), :]
        hent = _entropy(h, wc_ref, bc_ref)
        lam = 1.0 - hent / macc_ref[...]
        wgt = lam * h
        gid = gid_ref[0, 0, :]
        oh = (lax.broadcasted_iota(jnp.int32, (_G, _R), 0) == gid[None, :])
        pool_ref[...] += lax.dot_general(
            oh.astype(jnp.float32), wgt, (((1,), (0,)), ((), ())),
            preferred_element_type=jnp.float32)

        @pl.when(i == _NB - 1)
        def _fin():
            t_ref[...] = lax.dot_general(
                pool_ref[...], wc_ref[...], (((1,), (1,)), ((), ())),
                preferred_element_type=jnp.float32) + bc_ref[...]


def _tc_final(a2, W, b, g, be, Wc, bc, gid3):
    return pl.pallas_call(
        _final_body,
        grid=(3, _NB),
        in_specs=[
            pl.BlockSpec((1, _R, _D),
                         lambda p, i: (0, i * ((1 - p) * (2 - p) // 2), 0)),
            pl.BlockSpec((1, _R, _D),
                         lambda p, i: (1, i * ((1 - p) * (2 - p) // 2), 0)),
            pl.BlockSpec((_D, _D), lambda p, i: (0, 0)),
            pl.BlockSpec((1, _D), lambda p, i: (0, 0)),
            pl.BlockSpec((1, _D), lambda p, i: (0, 0)),
            pl.BlockSpec((1, _D), lambda p, i: (0, 0)),
            pl.BlockSpec((10, _D), lambda p, i: (0, 0)),
            pl.BlockSpec((1, 10), lambda p, i: (0, 0)),
            pl.BlockSpec((1, 1, _R), lambda p, i: (i * (p // 2), 0, 0)),
        ],
        out_specs=pl.BlockSpec((_G, 10), lambda p, i: (0, 0)),
        out_shape=jax.ShapeDtypeStruct((_G, 10), jnp.float32),
        scratch_shapes=[
            pltpu.VMEM((_N, _D), jnp.float32),
            pltpu.VMEM((2, _D), jnp.float32),
            pltpu.VMEM((1, 1), jnp.float32),
            pltpu.VMEM((_G, _D), jnp.float32),
        ],
    )(a2, a2, W, b, g, be, Wc, bc, gid3)


def kernel(x, edge_index, graph_ids, W0, b0, g0, be0, W1, b1, g1, be1,
           W2, b2, g2, be2, Wc, bc):
    pad = _E_PAD - _E
    pad_src = jnp.arange(pad, dtype=jnp.int32) * 997 % _N
    src2d = jnp.concatenate(
        [edge_index[0], pad_src]).reshape(_NW * _CPT, _CH)
    pad_dst = _DUMP + jnp.arange(pad, dtype=jnp.int32) % (_N_ACC - _N)
    dst2d = jnp.concatenate(
        [edge_index[1], pad_dst]).reshape(_NW * _CPT, _CH)
    zrows = jnp.zeros((_N_ACC, _D), jnp.float32)
    gid3 = graph_ids.reshape(_NB, 1, _R)
    bc2 = bc.reshape(1, 10)

    h = x
    for (W, b, gm, be) in ((W0, b0, g0, be0), (W1, b1, g1, be1)):
        a2 = _sc_scatter_sum(h, src2d, dst2d, zrows).reshape(_NC, _N_ACC, _D)
        h = _tc_layer(a2, W, b.reshape(1, _D),
                      gm.reshape(1, _D), be.reshape(1, _D))

    a2 = _sc_scatter_sum(h, src2d, dst2d, zrows).reshape(_NC, _N_ACC, _D)
    return _tc_final(a2, W2, b2.reshape(1, _D),
                     g2.reshape(1, _D), be2.reshape(1, _D), Wc, bc2, gid3)


# speedup vs baseline: 3.4363x; 1.0068x over previous
import functools

import jax
import jax.numpy as jnp
from jax import lax
from jax.experimental import pallas as pl
from jax.experimental.pallas import tpu as pltpu
from jax.experimental.pallas import tpu_sc as plsc

_N = 10000
_E = 320000
_D = 128
_G = 16

_NC = 2
_NS = 16
_NW = _NC * _NS
_CH = 64
_CPT = 160
_HC = 40
_NBUF = 4
_LG = 3
_EPT = _CPT * _CH
_E_PAD = _NW * _EPT
_N_ACC = 10112
_RPT = _N_ACC // _NS
_DUMP = _N

_R = 5000
_NB = _N // _R


def _sc_scatter_sum(h, src2d, dst2d, zrows):
    mesh = plsc.VectorSubcoreMesh(core_axis_name="c", subcore_axis_name="s")

    @functools.partial(
        pl.kernel,
        out_type=jax.ShapeDtypeStruct((_NC * _N_ACC, _D), jnp.float32),
        mesh=mesh,
        scratch_types=[
            pltpu.VMEM((_HC, _CH), jnp.int32),
            pltpu.VMEM((_HC, _CH), jnp.int32),
            pltpu.VMEM((_NBUF, _CH, _D), jnp.float32),
            pltpu.VMEM_SHARED((_N_ACC, _D), jnp.float32),
            pltpu.SemaphoreType.DMA,
            pltpu.SemaphoreType.DMA,
            pltpu.SemaphoreType.DMA,
            pltpu.SemaphoreType.DMA,
            pltpu.SemaphoreType.DMA,
            pltpu.SemaphoreType.DMA,
            pltpu.SemaphoreType.DMA,
            pltpu.SemaphoreType.DMA,
        ],
    )
    def k(h_hbm, src_hbm, dst_hbm, z_hbm, out_hbm,
          src_v, dst_v, rows_v, acc_sh,
          sem0, sem1, sem2, sem3, sem4, sem5, sem6, sem7):
        c = lax.axis_index("c")
        s = lax.axis_index("s")
        wid = c * _NS + s
        pltpu.sync_copy(z_hbm.at[pl.ds(s * _RPT, _RPT)],
                        acc_sh.at[pl.ds(s * _RPT, _RPT)])
        plsc.subcore_barrier()

        gsems = (sem0, sem1, sem2, sem3)
        ssems = (sem4, sem5, sem6, sem7)

        def gather(j, b):
            pltpu.async_copy(h_hbm.at[src_v.at[j]], rows_v.at[b], gsems[b])

        def gwait(j, b):
            pltpu.make_async_copy(
                h_hbm.at[src_v.at[j]], rows_v.at[b], gsems[b]).wait()

        def scat(j, b):
            pltpu.async_copy(rows_v.at[b], acc_sh.at[dst_v.at[j]],
                             ssems[b], add=True)

        def swait(j, b):
            pltpu.make_async_copy(
                rows_v.at[b], acc_sh.at[dst_v.at[j]], ssems[b]).wait()

        for half in range(_CPT // _HC):
            base = wid * _CPT + half * _HC
            pltpu.sync_copy(src_hbm.at[pl.ds(base, _HC)], src_v)
            pltpu.sync_copy(dst_hbm.at[pl.ds(base, _HC)], dst_v)

            def step(j, b):
                gwait(j, b)
                scat(j, b)
                nb = (b + _LG) % _NBUF
                swait(j + _LG - _NBUF, nb)
                gather(j + _LG, nb)

            for j in range(_LG):
                gather(j, j)
            for j in range(_NBUF):
                gwait(j, j)
                scat(j, j)
                if j + _LG < _NBUF:
                    gather(j + _LG, j + _LG)
                else:
                    swait(j + _LG - _NBUF, (j + _LG) % _NBUF)
                    gather(j + _LG, (j + _LG) % _NBUF)

            def body(g, carry):
                for b in range(_NBUF):
                    step(_NBUF * g + b, b)
                return carry

            lax.fori_loop(1, _HC // _NBUF - 1, body, 0)
            for j in range(_HC - _NBUF, _HC):
                gwait(j, j % _NBUF)
                scat(j, j % _NBUF)
                if j + _LG < _HC:
                    swait(j + _LG - _NBUF, (j + _LG) % _NBUF)
                    gather(j + _LG, (j + _LG) % _NBUF)
            for j in range(_HC - _NBUF, _HC):
                swait(j, j % _NBUF)

        plsc.subcore_barrier()
        row0 = c * _N_ACC + s * _RPT
        pltpu.sync_copy(acc_sh.at[pl.ds(s * _RPT, _RPT)],
                        out_hbm.at[pl.ds(row0, _RPT)])

    return k(h, src2d, dst2d, zrows)


def _bn_elu(z, acc_ref, g_ref, be_ref):
    mu = acc_ref[0:1, :] * (1.0 / _N)
    var = acc_ref[1:2, :] * (1.0 / _N) - mu * mu
    inv = lax.rsqrt(var + 1e-5)
    y = (z - mu) * (inv * g_ref[...]) + be_ref[...]
    return jnp.where(y > 0, y, jnp.exp(y) - 1.0)


def _layer_body(a0_ref, a1_ref, w_ref, b_ref, g_ref, be_ref,
                h_ref, z_sc, acc_ref):
    p = pl.program_id(0)
    i = pl.program_id(1)

    @pl.when(p == 0)
    def _compute():
        a = a0_ref[0] + a1_ref[0]
        z = lax.dot_general(a, w_ref[...], (((1,), (1,)), ((), ())),
                            preferred_element_type=jnp.float32) + b_ref[...]
        z_sc[pl.ds(i * _R, _R), :] = z

        @pl.when(i == 0)
        def _init():
            acc_ref[...] = jnp.zeros_like(acc_ref)

        acc_ref[0:1, :] += jnp.sum(z, axis=0, keepdims=True)
        acc_ref[1:2, :] += jnp.sum(z * z, axis=0, keepdims=True)

    @pl.when(p == 1)
    def _norm():
        z = z_sc[pl.ds(i * _R, _R), :]
        h_ref[...] = _bn_elu(z, acc_ref, g_ref, be_ref)


def _tc_layer(a2, W, b, g, be):
    return pl.pallas_call(
        _layer_body,
        grid=(2, _NB),
        in_specs=[
            pl.BlockSpec((1, _R, _D), lambda p, i: (0, i * (1 - p), 0)),
            pl.BlockSpec((1, _R, _D), lambda p, i: (1, i * (1 - p), 0)),
            pl.BlockSpec((_D, _D), lambda p, i: (0, 0)),
            pl.BlockSpec((1, _D), lambda p, i: (0, 0)),
            pl.BlockSpec((1, _D), lambda p, i: (0, 0)),
            pl.BlockSpec((1, _D), lambda p, i: (0, 0)),
        ],
        out_specs=pl.BlockSpec((_R, _D), lambda p, i: (i * p, 0)),
        out_shape=jax.ShapeDtypeStruct((_N, _D), jnp.float32),
        scratch_shapes=[
            pltpu.VMEM((_N, _D), jnp.float32),
            pltpu.VMEM((2, _D), jnp.float32),
        ],
    )(a2, a2, W, b, g, be)


def _entropy(h, wc_ref, bc_ref):
    logits = lax.dot_general(h, wc_ref[...], (((1,), (1,)), ((), ())),
                             preferred_element_type=jnp.float32) + bc_ref[...]
    m = jnp.max(logits, axis=1, keepdims=True)
    lse = m + jnp.log(jnp.sum(jnp.exp(logits - m), axis=1, keepdims=True))
    logp = logits - lse
    return -jnp.sum(jnp.exp(logp) * logp, axis=1, keepdims=True)


def _final_body(a0_ref, a1_ref, w_ref, b_ref, g_ref, be_ref, wc_ref, bc_ref,
                gid_ref, t_ref, z_sc, acc_ref, macc_ref, pool_ref):
    p = pl.program_id(0)
    i = pl.program_id(1)

    @pl.when(p == 0)
    def _compute():
        a = a0_ref[0] + a1_ref[0]
        z = lax.dot_general(a, w_ref[...], (((1,), (1,)), ((), ())),
                            preferred_element_type=jnp.float32) + b_ref[...]
        z_sc[pl.ds(i * _R, _R), :] = z

        @pl.when(i == 0)
        def _init():
            acc_ref[...] = jnp.zeros_like(acc_ref)
            macc_ref[...] = jnp.full((1, 1), -jnp.inf, jnp.float32)
            pool_ref[...] = jnp.zeros_like(pool_ref)

        acc_ref[0:1, :] += jnp.sum(z, axis=0, keepdims=True)
        acc_ref[1:2, :] += jnp.sum(z * z, axis=0, keepdims=True)

    @pl.when(p == 1)
    def _norm():
        z = z_sc[pl.ds(i * _R, _R), :]
        h = _bn_elu(z, acc_ref, g_ref, be_ref)
        z_sc[pl.ds(i * _R, _R), :] = h
        hent = _entropy(h, wc_ref, bc_ref)
        macc_ref[...] = jnp.maximum(macc_ref[...],
                                    jnp.max(hent, axis=0, keepdims=True))

    @pl.when(p == 2)
    def _head():
        h = z_sc[pl.ds(i * _R, _R), :]
        hent = _entropy(h, wc_ref, bc_ref)
        lam = 1.0 - hent / macc_ref[...]
        wgt = lam * h
        gid = gid_ref[0, 0, :]
        oh = (lax.broadcasted_iota(jnp.int32, (_G, _R), 0) == gid[None, :])
        pool_ref[...] += lax.dot_general(
            oh.astype(jnp.float32), wgt, (((1,), (0,)), ((), ())),
            preferred_element_type=jnp.float32)

        @pl.when(i == _NB - 1)
        def _fin():
            t_ref[...] = lax.dot_general(
                pool_ref[...], wc_ref[...], (((1,), (1,)), ((), ())),
                preferred_element_type=jnp.float32) + bc_ref[...]


def _tc_final(a2, W, b, g, be, Wc, bc, gid3):
    return pl.pallas_call(
        _final_body,
        grid=(3, _NB),
        in_specs=[
            pl.BlockSpec((1, _R, _D),
                         lambda p, i: (0, i * ((1 - p) * (2 - p) // 2), 0)),
            pl.BlockSpec((1, _R, _D),
                         lambda p, i: (1, i * ((1 - p) * (2 - p) // 2), 0)),
            pl.BlockSpec((_D, _D), lambda p, i: (0, 0)),
            pl.BlockSpec((1, _D), lambda p, i: (0, 0)),
            pl.BlockSpec((1, _D), lambda p, i: (0, 0)),
            pl.BlockSpec((1, _D), lambda p, i: (0, 0)),
            pl.BlockSpec((10, _D), lambda p, i: (0, 0)),
            pl.BlockSpec((1, 10), lambda p, i: (0, 0)),
            pl.BlockSpec((1, 1, _R), lambda p, i: (i * (p // 2), 0, 0)),
        ],
        out_specs=pl.BlockSpec((_G, 10), lambda p, i: (0, 0)),
        out_shape=jax.ShapeDtypeStruct((_G, 10), jnp.float32),
        scratch_shapes=[
            pltpu.VMEM((_N, _D), jnp.float32),
            pltpu.VMEM((2, _D), jnp.float32),
            pltpu.VMEM((1, 1), jnp.float32),
            pltpu.VMEM((_G, _D), jnp.float32),
        ],
    )(a2, a2, W, b, g, be, Wc, bc, gid3)


def kernel(x, edge_index, graph_ids, W0, b0, g0, be0, W1, b1, g1, be1,
           W2, b2, g2, be2, Wc, bc):
    pad = _E_PAD - _E
    pad_src = jnp.arange(pad, dtype=jnp.int32) * 997 % _N
    src2d = jnp.concatenate(
        [edge_index[0], pad_src]).reshape(_NW * _CPT, _CH)
    pad_dst = _DUMP + jnp.arange(pad, dtype=jnp.int32) % (_N_ACC - _N)
    dst2d = jnp.concatenate(
        [edge_index[1], pad_dst]).reshape(_NW * _CPT, _CH)
    zrows = jnp.zeros((_N_ACC, _D), jnp.float32)
    gid3 = graph_ids.reshape(_NB, 1, _R)
    bc2 = bc.reshape(1, 10)

    h = x
    for (W, b, gm, be) in ((W0, b0, g0, be0), (W1, b1, g1, be1)):
        a2 = _sc_scatter_sum(h, src2d, dst2d, zrows).reshape(_NC, _N_ACC, _D)
        h = _tc_layer(a2, W, b.reshape(1, _D),
                      gm.reshape(1, _D), be.reshape(1, _D))

    a2 = _sc_scatter_sum(h, src2d, dst2d, zrows).reshape(_NC, _N_ACC, _D)
    return _tc_final(a2, W2, b2.reshape(1, _D),
                     g2.reshape(1, _D), be2.reshape(1, _D), Wc, bc2, gid3)


# speedup vs baseline: 3.4678x; 1.0092x over previous
import functools

import jax
import jax.numpy as jnp
from jax import lax
from jax.experimental import pallas as pl
from jax.experimental.pallas import tpu as pltpu
from jax.experimental.pallas import tpu_sc as plsc

_N = 10000
_E = 320000
_D = 128
_G = 16

_NC = 2
_NS = 16
_NW = _NC * _NS
_CH = 64
_CPT = 160
_HC = 40
_NBUF = 4
_LG = 3
_EPT = _CPT * _CH
_E_PAD = _NW * _EPT
_N_ACC = 10112
_RPT = _N_ACC // _NS
_DUMP = _N

_R = 10000
_NB = _N // _R


def _sc_scatter_sum(h, src2d, dst2d, zrows):
    mesh = plsc.VectorSubcoreMesh(core_axis_name="c", subcore_axis_name="s")

    @functools.partial(
        pl.kernel,
        out_type=jax.ShapeDtypeStruct((_NC * _N_ACC, _D), jnp.float32),
        mesh=mesh,
        scratch_types=[
            pltpu.VMEM((_HC, _CH), jnp.int32),
            pltpu.VMEM((_HC, _CH), jnp.int32),
            pltpu.VMEM((_NBUF, _CH, _D), jnp.float32),
            pltpu.VMEM_SHARED((_N_ACC, _D), jnp.float32),
            pltpu.SemaphoreType.DMA,
            pltpu.SemaphoreType.DMA,
            pltpu.SemaphoreType.DMA,
            pltpu.SemaphoreType.DMA,
            pltpu.SemaphoreType.DMA,
            pltpu.SemaphoreType.DMA,
            pltpu.SemaphoreType.DMA,
            pltpu.SemaphoreType.DMA,
        ],
    )
    def k(h_hbm, src_hbm, dst_hbm, z_hbm, out_hbm,
          src_v, dst_v, rows_v, acc_sh,
          sem0, sem1, sem2, sem3, sem4, sem5, sem6, sem7):
        c = lax.axis_index("c")
        s = lax.axis_index("s")
        wid = c * _NS + s
        pltpu.sync_copy(z_hbm.at[pl.ds(s * _RPT, _RPT)],
                        acc_sh.at[pl.ds(s * _RPT, _RPT)])
        plsc.subcore_barrier()

        gsems = (sem0, sem1, sem2, sem3)
        ssems = (sem4, sem5, sem6, sem7)

        def gather(j, b):
            pltpu.async_copy(h_hbm.at[src_v.at[j]], rows_v.at[b], gsems[b])

        def gwait(j, b):
            pltpu.make_async_copy(
                h_hbm.at[src_v.at[j]], rows_v.at[b], gsems[b]).wait()

        def scat(j, b):
            pltpu.async_copy(rows_v.at[b], acc_sh.at[dst_v.at[j]],
                             ssems[b], add=True)

        def swait(j, b):
            pltpu.make_async_copy(
                rows_v.at[b], acc_sh.at[dst_v.at[j]], ssems[b]).wait()

        for half in range(_CPT // _HC):
            base = wid * _CPT + half * _HC
            pltpu.sync_copy(src_hbm.at[pl.ds(base, _HC)], src_v)
            pltpu.sync_copy(dst_hbm.at[pl.ds(base, _HC)], dst_v)

            def step(j, b):
                gwait(j, b)
                scat(j, b)
                nb = (b + _LG) % _NBUF
                swait(j + _LG - _NBUF, nb)
                gather(j + _LG, nb)

            for j in range(_LG):
                gather(j, j)
            for j in range(_NBUF):
                gwait(j, j)
                scat(j, j)
                if j + _LG < _NBUF:
                    gather(j + _LG, j + _LG)
                else:
                    swait(j + _LG - _NBUF, (j + _LG) % _NBUF)
                    gather(j + _LG, (j + _LG) % _NBUF)

            def body(g, carry):
                for b in range(_NBUF):
                    step(_NBUF * g + b, b)
                return carry

            lax.fori_loop(1, _HC // _NBUF - 1, body, 0)
            for j in range(_HC - _NBUF, _HC):
                gwait(j, j % _NBUF)
                scat(j, j % _NBUF)
                if j + _LG < _HC:
                    swait(j + _LG - _NBUF, (j + _LG) % _NBUF)
                    gather(j + _LG, (j + _LG) % _NBUF)
            for j in range(_HC - _NBUF, _HC):
                swait(j, j % _NBUF)

        plsc.subcore_barrier()
        row0 = c * _N_ACC + s * _RPT
        pltpu.sync_copy(acc_sh.at[pl.ds(s * _RPT, _RPT)],
                        out_hbm.at[pl.ds(row0, _RPT)])

    return k(h, src2d, dst2d, zrows)


def _bn_elu(z, acc_ref, g_ref, be_ref):
    mu = acc_ref[0:1, :] * (1.0 / _N)
    var = acc_ref[1:2, :] * (1.0 / _N) - mu * mu
    inv = lax.rsqrt(var + 1e-5)
    y = (z - mu) * (inv * g_ref[...]) + be_ref[...]
    return jnp.where(y > 0, y, jnp.exp(y) - 1.0)


def _layer_body(a0_ref, a1_ref, w_ref, b_ref, g_ref, be_ref,
                h_ref, z_sc, acc_ref):
    p = pl.program_id(0)
    i = pl.program_id(1)

    @pl.when(p == 0)
    def _compute():
        a = a0_ref[0] + a1_ref[0]
        z = lax.dot_general(a, w_ref[...], (((1,), (1,)), ((), ())),
                            preferred_element_type=jnp.float32) + b_ref[...]
        z_sc[pl.ds(i * _R, _R), :] = z

        @pl.when(i == 0)
        def _init():
            acc_ref[...] = jnp.zeros_like(acc_ref)

        acc_ref[0:1, :] += jnp.sum(z, axis=0, keepdims=True)
        acc_ref[1:2, :] += jnp.sum(z * z, axis=0, keepdims=True)

    @pl.when(p == 1)
    def _norm():
        z = z_sc[pl.ds(i * _R, _R), :]
        h_ref[...] = _bn_elu(z, acc_ref, g_ref, be_ref)


def _tc_layer(a2, W, b, g, be):
    return pl.pallas_call(
        _layer_body,
        grid=(2, _NB),
        in_specs=[
            pl.BlockSpec((1, _R, _D), lambda p, i: (0, i * (1 - p), 0)),
            pl.BlockSpec((1, _R, _D), lambda p, i: (1, i * (1 - p), 0)),
            pl.BlockSpec((_D, _D), lambda p, i: (0, 0)),
            pl.BlockSpec((1, _D), lambda p, i: (0, 0)),
            pl.BlockSpec((1, _D), lambda p, i: (0, 0)),
            pl.BlockSpec((1, _D), lambda p, i: (0, 0)),
        ],
        out_specs=pl.BlockSpec((_R, _D), lambda p, i: (i * p, 0)),
        out_shape=jax.ShapeDtypeStruct((_N, _D), jnp.float32),
        scratch_shapes=[
            pltpu.VMEM((_N, _D), jnp.float32),
            pltpu.VMEM((2, _D), jnp.float32),
        ],
    )(a2, a2, W, b, g, be)


def _entropy(h, wc_ref, bc_ref):
    logits = lax.dot_general(h, wc_ref[...], (((1,), (1,)), ((), ())),
                             preferred_element_type=jnp.float32) + bc_ref[...]
    m = jnp.max(logits, axis=1, keepdims=True)
    lse = m + jnp.log(jnp.sum(jnp.exp(logits - m), axis=1, keepdims=True))
    logp = logits - lse
    return -jnp.sum(jnp.exp(logp) * logp, axis=1, keepdims=True)


def _final_body(a0_ref, a1_ref, w_ref, b_ref, g_ref, be_ref, wc_ref, bc_ref,
                gid_ref, t_ref, z_sc, acc_ref, macc_ref, pool_ref):
    p = pl.program_id(0)
    i = pl.program_id(1)

    @pl.when(p == 0)
    def _compute():
        a = a0_ref[0] + a1_ref[0]
        z = lax.dot_general(a, w_ref[...], (((1,), (1,)), ((), ())),
                            preferred_element_type=jnp.float32) + b_ref[...]
        z_sc[pl.ds(i * _R, _R), :] = z

        @pl.when(i == 0)
        def _init():
            acc_ref[...] = jnp.zeros_like(acc_ref)
            macc_ref[...] = jnp.full((1, 1), -jnp.inf, jnp.float32)
            pool_ref[...] = jnp.zeros_like(pool_ref)

        acc_ref[0:1, :] += jnp.sum(z, axis=0, keepdims=True)
        acc_ref[1:2, :] += jnp.sum(z * z, axis=0, keepdims=True)

    @pl.when(p == 1)
    def _norm():
        z = z_sc[pl.ds(i * _R, _R), :]
        h = _bn_elu(z, acc_ref, g_ref, be_ref)
        z_sc[pl.ds(i * _R, _R), :] = h
        hent = _entropy(h, wc_ref, bc_ref)
        macc_ref[...] = jnp.maximum(macc_ref[...],
                                    jnp.max(hent, axis=0, keepdims=True))

    @pl.when(p == 2)
    def _head():
        h = z_sc[pl.ds(i * _R, _R), :]
        hent = _entropy(h, wc_ref, bc_ref)
        lam = 1.0 - hent / macc_ref[...]
        wgt = lam * h
        gid = gid_ref[0, 0, :]
        oh = (lax.broadcasted_iota(jnp.int32, (_G, _R), 0) == gid[None, :])
        pool_ref[...] += lax.dot_general(
            oh.astype(jnp.float32), wgt, (((1,), (0,)), ((), ())),
            preferred_element_type=jnp.float32)

        @pl.when(i == _NB - 1)
        def _fin():
            t_ref[...] = lax.dot_general(
                pool_ref[...], wc_ref[...], (((1,), (1,)), ((), ())),
                preferred_element_type=jnp.float32) + bc_ref[...]


def _tc_final(a2, W, b, g, be, Wc, bc, gid3):
    return pl.pallas_call(
        _final_body,
        grid=(3, _NB),
        in_specs=[
            pl.BlockSpec((1, _R, _D),
                         lambda p, i: (0, i * ((1 - p) * (2 - p) // 2), 0)),
            pl.BlockSpec((1, _R, _D),
                         lambda p, i: (1, i * ((1 - p) * (2 - p) // 2), 0)),
            pl.BlockSpec((_D, _D), lambda p, i: (0, 0)),
            pl.BlockSpec((1, _D), lambda p, i: (0, 0)),
            pl.BlockSpec((1, _D), lambda p, i: (0, 0)),
            pl.BlockSpec((1, _D), lambda p, i: (0, 0)),
            pl.BlockSpec((10, _D), lambda p, i: (0, 0)),
            pl.BlockSpec((1, 10), lambda p, i: (0, 0)),
            pl.BlockSpec((1, 1, _R), lambda p, i: (i * (p // 2), 0, 0)),
        ],
        out_specs=pl.BlockSpec((_G, 10), lambda p, i: (0, 0)),
        out_shape=jax.ShapeDtypeStruct((_G, 10), jnp.float32),
        scratch_shapes=[
            pltpu.VMEM((_N, _D), jnp.float32),
            pltpu.VMEM((2, _D), jnp.float32),
            pltpu.VMEM((1, 1), jnp.float32),
            pltpu.VMEM((_G, _D), jnp.float32),
        ],
    )(a2, a2, W, b, g, be, Wc, bc, gid3)


def kernel(x, edge_index, graph_ids, W0, b0, g0, be0, W1, b1, g1, be1,
           W2, b2, g2, be2, Wc, bc):
    pad = _E_PAD - _E
    pad_src = jnp.arange(pad, dtype=jnp.int32) * 997 % _N
    src2d = jnp.concatenate(
        [edge_index[0], pad_src]).reshape(_NW * _CPT, _CH)
    pad_dst = _DUMP + jnp.arange(pad, dtype=jnp.int32) % (_N_ACC - _N)
    dst2d = jnp.concatenate(
        [edge_index[1], pad_dst]).reshape(_NW * _CPT, _CH)
    zrows = jnp.zeros((_N_ACC, _D), jnp.float32)
    gid3 = graph_ids.reshape(_NB, 1, _R)
    bc2 = bc.reshape(1, 10)

    h = x
    for (W, b, gm, be) in ((W0, b0, g0, be0), (W1, b1, g1, be1)):
        a2 = _sc_scatter_sum(h, src2d, dst2d, zrows).reshape(_NC, _N_ACC, _D)
        h = _tc_layer(a2, W, b.reshape(1, _D),
                      gm.reshape(1, _D), be.reshape(1, _D))

    a2 = _sc_scatter_sum(h, src2d, dst2d, zrows).reshape(_NC, _N_ACC, _D)
    return _tc_final(a2, W2, b2.reshape(1, _D),
                     g2.reshape(1, _D), be2.reshape(1, _D), Wc, bc2, gid3)
